# Initial kernel scaffold; baseline (speedup 1.0000x reference)
import jax
import jax.numpy as jnp
from jax.experimental import pallas as pl


def kernel(x, edge_index, batch, params):
    raise NotImplementedError("write your pallas kernel here")


# speedup vs baseline: 6.2834x; 6.2834x over previous
import functools

import jax
import jax.numpy as jnp
from jax import lax
from jax.experimental import pallas as pl
from jax.experimental.pallas import tpu as pltpu
from jax.experimental.pallas import tpu_sc as plsc

N = 10000
E = 320000
F = 128
G = 128
BN_EPS = 1e-5

NC = 2
NS = 16
NW = NC * NS

K = 125
CHUNKS = E // (NW * K)
RPT = 624
TAIL = N - RPT * NS
ZCH = 104

BM = 400
NB = N // BM
PB = 200
NPB = N // PB


def _sc_agg_body(h_hbm, src_hbm, dst_hbm, out_hbm, src_v, dst_v, rows_v, acc_sh):
    cid = lax.axis_index("c")
    sid = lax.axis_index("s")
    wid = sid * NC + cid

    def _zero_row(i, _):
        for j in range(F // 16):
            rows_v[i, pl.ds(j * 16, 16)] = jnp.zeros((16,), jnp.float32)
        return 0
    lax.fori_loop(0, ZCH, _zero_row, 0)
    for r in range(RPT // ZCH):
        pltpu.sync_copy(rows_v.at[pl.ds(0, ZCH)],
                        acc_sh.at[pl.ds(sid * RPT + r * ZCH, ZCH)])

    @pl.when(sid == NS - 1)
    def _():
        pltpu.sync_copy(rows_v.at[pl.ds(0, TAIL)],
                        acc_sh.at[pl.ds(N - TAIL, TAIL)])

    plsc.subcore_barrier()

    pltpu.sync_copy(src_hbm.at[wid], src_v)
    pltpu.sync_copy(dst_hbm.at[wid], dst_v)

    def _chunk(j, _):
        pltpu.sync_copy(h_hbm.at[src_v.at[j]], rows_v)
        pltpu.sync_copy(rows_v, acc_sh.at[dst_v.at[j]], add=True)
        return 0
    lax.fori_loop(0, CHUNKS, _chunk, 0)

    plsc.subcore_barrier()
    pltpu.sync_copy(acc_sh.at[pl.ds(sid * RPT, RPT)],
                    out_hbm.at[cid, pl.ds(sid * RPT, RPT)])

    @pl.when(sid == NS - 1)
    def _():
        pltpu.sync_copy(acc_sh.at[pl.ds(N - TAIL, TAIL)],
                        out_hbm.at[cid, pl.ds(N - TAIL, TAIL)])


@functools.lru_cache(maxsize=None)
def _get_sc_agg():
    return pl.kernel(
        _sc_agg_body,
        out_type=jax.ShapeDtypeStruct((NC, N, F), jnp.float32),
        mesh=plsc.VectorSubcoreMesh(core_axis_name="c", subcore_axis_name="s"),
        scratch_types=[
            pltpu.VMEM((CHUNKS, K), jnp.int32),
            pltpu.VMEM((CHUNKS, K), jnp.int32),
            pltpu.VMEM((K, F), jnp.float32),
            pltpu.VMEM_SHARED((N, F), jnp.float32),
        ],
    )


def _sc_agg(h, src, dst):
    return _get_sc_agg()(h, src, dst)


def _mlp_body(h_ref, a0_ref, a1_ref, w1_ref, b1_ref, w2_ref, b2_ref,
              t_ref, s1_ref, s2_ref):
    z = h_ref[...] + a0_ref[...] + a1_ref[...]
    u = jnp.maximum(
        jnp.dot(z, w1_ref[...], preferred_element_type=jnp.float32) + b1_ref[...],
        0.0)
    t = jnp.maximum(
        jnp.dot(u, w2_ref[...], preferred_element_type=jnp.float32) + b2_ref[...],
        0.0)
    t_ref[...] = t

    i = pl.program_id(0)
    mb = jnp.mean(t, axis=0, keepdims=True)
    m2b = jnp.sum((t - mb) * (t - mb), axis=0, keepdims=True)

    @pl.when(i == 0)
    def _():
        s1_ref[...] = mb
        s2_ref[...] = m2b

    @pl.when(i > 0)
    def _():
        na = i.astype(jnp.float32) * BM
        n = na + BM
        delta = mb - s1_ref[...]
        s1_ref[...] += delta * (BM / n)
        s2_ref[...] += m2b + delta * delta * (na * BM / n)


def _mlp(h, agg, p):
    return pl.pallas_call(
        _mlp_body,
        grid=(NB,),
        in_specs=[
            pl.BlockSpec((BM, F), lambda i: (i, 0)),
            pl.BlockSpec((BM, F), lambda i: (i, 0)),
            pl.BlockSpec((BM, F), lambda i: (i, 0)),
            pl.BlockSpec((F, F), lambda i: (0, 0)),
            pl.BlockSpec((1, F), lambda i: (0, 0)),
            pl.BlockSpec((F, F), lambda i: (0, 0)),
            pl.BlockSpec((1, F), lambda i: (0, 0)),
        ],
        out_specs=[
            pl.BlockSpec((BM, F), lambda i: (i, 0)),
            pl.BlockSpec((1, F), lambda i: (0, 0)),
            pl.BlockSpec((1, F), lambda i: (0, 0)),
        ],
        out_shape=[
            jax.ShapeDtypeStruct((N, F), jnp.float32),
            jax.ShapeDtypeStruct((1, F), jnp.float32),
            jax.ShapeDtypeStruct((1, F), jnp.float32),
        ],
    )(h, agg[0], agg[1], p["W1"], p["b1"].reshape(1, F), p["W2"],
      p["b2"].reshape(1, F))


def _norm_body(t_ref, s1_ref, s2_ref, g_ref, be_ref, o_ref):
    mean = s1_ref[...]
    var = s2_ref[...] / N
    a = g_ref[...] * lax.rsqrt(var + BN_EPS)
    c = be_ref[...] - mean * a
    o_ref[...] = t_ref[...] * a + c


def _norm(t, s1, s2, gamma, beta):
    return pl.pallas_call(
        _norm_body,
        grid=(NB,),
        in_specs=[
            pl.BlockSpec((BM, F), lambda i: (i, 0)),
            pl.BlockSpec((1, F), lambda i: (0, 0)),
            pl.BlockSpec((1, F), lambda i: (0, 0)),
            pl.BlockSpec((1, F), lambda i: (0, 0)),
            pl.BlockSpec((1, F), lambda i: (0, 0)),
        ],
        out_specs=pl.BlockSpec((BM, F), lambda i: (i, 0)),
        out_shape=jax.ShapeDtypeStruct((N, F), jnp.float32),
    )(t, s1, s2, gamma.reshape(1, F), beta.reshape(1, F))


def _pool_body(h_ref, bat_ref, w1m_ref, b1m_ref, w2m_ref, b2m_ref,
               w1v_ref, b1v_ref, w2v_ref, b2v_ref,
               mu_ref, lv_ref, pooled, cnt):
    i = pl.program_id(0)
    bat = bat_ref[...].reshape(1, PB)
    seg = lax.broadcasted_iota(jnp.int32, (G, PB), 0)
    onehot = jnp.where(bat == seg, 1.0, 0.0)
    pt = lax.dot_general(onehot, h_ref[...], (((1,), (0,)), ((), ())),
                         preferred_element_type=jnp.float32)
    pc = jnp.sum(onehot, axis=1, keepdims=True)

    @pl.when(i == 0)
    def _():
        pooled[...] = jnp.zeros_like(pooled)
        cnt[...] = jnp.zeros_like(cnt)

    pooled[...] += pt
    cnt[...] += pc

    @pl.when(i == NPB - 1)
    def _():
        g = pooled[...] * (1.0 / jnp.maximum(cnt[...], 1.0))
        um = jnp.maximum(
            jnp.dot(g, w1m_ref[...], preferred_element_type=jnp.float32)
            + b1m_ref[...], 0.0)
        mu_ref[...] = jnp.maximum(
            jnp.dot(um, w2m_ref[...], preferred_element_type=jnp.float32)
            + b2m_ref[...], 0.0)
        uv = jnp.maximum(
            jnp.dot(g, w1v_ref[...], preferred_element_type=jnp.float32)
            + b1v_ref[...], 0.0)
        lv_ref[...] = jnp.maximum(
            jnp.dot(uv, w2v_ref[...], preferred_element_type=jnp.float32)
            + b2v_ref[...], 0.0)


def _pool_classify(h, batf, pm, pv):
    return pl.pallas_call(
        _pool_body,
        grid=(NPB,),
        in_specs=[
            pl.BlockSpec((PB, F), lambda i: (i, 0)),
            pl.BlockSpec((1, 1, PB), lambda i: (i, 0, 0)),
            pl.BlockSpec((F, F), lambda i: (0, 0)),
            pl.BlockSpec((1, F), lambda i: (0, 0)),
            pl.BlockSpec((F, F), lambda i: (0, 0)),
            pl.BlockSpec((1, F), lambda i: (0, 0)),
            pl.BlockSpec((F, F), lambda i: (0, 0)),
            pl.BlockSpec((1, F), lambda i: (0, 0)),
            pl.BlockSpec((F, F), lambda i: (0, 0)),
            pl.BlockSpec((1, F), lambda i: (0, 0)),
        ],
        out_specs=[
            pl.BlockSpec((G, F), lambda i: (0, 0)),
            pl.BlockSpec((G, F), lambda i: (0, 0)),
        ],
        out_shape=[
            jax.ShapeDtypeStruct((G, F), jnp.float32),
            jax.ShapeDtypeStruct((G, F), jnp.float32),
        ],
        scratch_shapes=[
            pltpu.VMEM((G, F), jnp.float32),
            pltpu.VMEM((G, 1), jnp.float32),
        ],
    )(h, batf,
      pm["W1"], pm["b1"].reshape(1, F), pm["W2"], pm["b2"].reshape(1, F),
      pv["W1"], pv["b1"].reshape(1, F), pv["W2"], pv["b2"].reshape(1, F))


def kernel(x, edge_index, batch, params):
    src = edge_index[0].astype(jnp.int32).reshape(NW, CHUNKS, K)
    dst = edge_index[1].astype(jnp.int32).reshape(NW, CHUNKS, K)
    batf = batch.astype(jnp.int32).reshape(NPB, 1, PB)

    h = x
    for i in range(3):
        p = params["convs"][i]
        agg = _sc_agg(h, src, dst)
        t, s1, s2 = _mlp(h, agg, p)
        h = _norm(t, s1, s2, p["gamma"], p["beta"])

    agg = _sc_agg(h, src, dst)
    p3 = params["convs"][3]
    p4 = params["convs"][4]
    t_mu, s1m, s2m = _mlp(h, agg, p3)
    t_lv, s1v, s2v = _mlp(h, agg, p4)
    node_mu = _norm(t_mu, s1m, s2m, p3["gamma"], p3["beta"])
    node_logv = _norm(t_lv, s1v, s2v, p4["gamma"], p4["beta"])

    class_mu, class_logv = _pool_classify(h, batf, params["cls_mu"],
                                          params["cls_logv"])
    return (node_mu, node_logv, class_mu, class_logv)


# speedup vs baseline: 9.0700x; 1.4435x over previous
import functools

import jax
import jax.numpy as jnp
from jax import lax
from jax.experimental import pallas as pl
from jax.experimental.pallas import tpu as pltpu
from jax.experimental.pallas import tpu_sc as plsc

N = 10000
E = 320000
F = 128
G = 128
BN_EPS = 1e-5

NC = 2
NS = 16
NW = NC * NS

K = 125
CHUNKS = E // (NW * K)
HC = CHUNKS // 2
RPT = 624
TAIL = N - RPT * NS
ZCH = 104

BM = 400
NB = N // BM
PB = 200
NPB = N // PB


def _sc_agg_body(h_hbm, src_hbm, dst_hbm, out_hbm, src_v, dst_v, rows_v, rows_b,
                 acc_sh, sem0, sem1):
    cid = lax.axis_index("c")
    sid = lax.axis_index("s")
    wid = sid * NC + cid

    def _zero_row(i, _):
        for j in range(F // 16):
            rows_v[i, pl.ds(j * 16, 16)] = jnp.zeros((16,), jnp.float32)
        return 0
    lax.fori_loop(0, ZCH, _zero_row, 0)
    for r in range(RPT // ZCH):
        pltpu.sync_copy(rows_v.at[pl.ds(0, ZCH)],
                        acc_sh.at[pl.ds(sid * RPT + r * ZCH, ZCH)])

    @pl.when(sid == NS - 1)
    def _():
        pltpu.sync_copy(rows_v.at[pl.ds(0, TAIL)],
                        acc_sh.at[pl.ds(N - TAIL, TAIL)])

    plsc.subcore_barrier()

    rows = (rows_v, rows_b)
    sems = (sem0, sem1)
    for half in range(2):
        pltpu.sync_copy(src_hbm.at[wid, pl.ds(half * HC, HC)], src_v)
        pltpu.sync_copy(dst_hbm.at[wid, pl.ds(half * HC, HC)], dst_v)
        pltpu.async_copy(h_hbm.at[src_v.at[0]], rows[0], sems[0])
        pltpu.async_copy(h_hbm.at[src_v.at[1]], rows[1], sems[1])

        def _pair(jj, _):
            j = 2 * jj
            for b in range(2):
                pltpu.make_async_copy(h_hbm.at[src_v.at[0]], rows[b],
                                      sems[b]).wait()
                pltpu.sync_copy(rows[b], acc_sh.at[dst_v.at[j + b]], add=True)

                @pl.when(j + b + 2 < HC)
                def _():
                    pltpu.async_copy(h_hbm.at[src_v.at[j + b + 2]], rows[b],
                                     sems[b])
            return 0
        lax.fori_loop(0, HC // 2, _pair, 0)

    plsc.subcore_barrier()
    pltpu.sync_copy(acc_sh.at[pl.ds(sid * RPT, RPT)],
                    out_hbm.at[cid, pl.ds(sid * RPT, RPT)])

    @pl.when(sid == NS - 1)
    def _():
        pltpu.sync_copy(acc_sh.at[pl.ds(N - TAIL, TAIL)],
                        out_hbm.at[cid, pl.ds(N - TAIL, TAIL)])


@functools.lru_cache(maxsize=None)
def _get_sc_agg():
    return pl.kernel(
        _sc_agg_body,
        out_type=jax.ShapeDtypeStruct((NC, N, F), jnp.float32),
        mesh=plsc.VectorSubcoreMesh(core_axis_name="c", subcore_axis_name="s"),
        scratch_types=[
            pltpu.VMEM((HC, K), jnp.int32),
            pltpu.VMEM((HC, K), jnp.int32),
            pltpu.VMEM((K, F), jnp.float32),
            pltpu.VMEM((K, F), jnp.float32),
            pltpu.VMEM_SHARED((N, F), jnp.float32),
            pltpu.SemaphoreType.DMA,
            pltpu.SemaphoreType.DMA,
        ],
    )


def _sc_agg(h, src, dst):
    return _get_sc_agg()(h, src, dst)


def _mlp_body(h_ref, a0_ref, a1_ref, w1_ref, b1_ref, w2_ref, b2_ref,
              t_ref, s1_ref, s2_ref):
    z = h_ref[...] + a0_ref[...] + a1_ref[...]
    u = jnp.maximum(
        jnp.dot(z, w1_ref[...], preferred_element_type=jnp.float32) + b1_ref[...],
        0.0)
    t = jnp.maximum(
        jnp.dot(u, w2_ref[...], preferred_element_type=jnp.float32) + b2_ref[...],
        0.0)
    t_ref[...] = t

    i = pl.program_id(0)
    mb = jnp.mean(t, axis=0, keepdims=True)
    m2b = jnp.sum((t - mb) * (t - mb), axis=0, keepdims=True)

    @pl.when(i == 0)
    def _():
        s1_ref[...] = mb
        s2_ref[...] = m2b

    @pl.when(i > 0)
    def _():
        na = i.astype(jnp.float32) * BM
        n = na + BM
        delta = mb - s1_ref[...]
        s1_ref[...] += delta * (BM / n)
        s2_ref[...] += m2b + delta * delta * (na * BM / n)


def _mlp(h, agg, p):
    return pl.pallas_call(
        _mlp_body,
        grid=(NB,),
        in_specs=[
            pl.BlockSpec((BM, F), lambda i: (i, 0)),
            pl.BlockSpec((BM, F), lambda i: (i, 0)),
            pl.BlockSpec((BM, F), lambda i: (i, 0)),
            pl.BlockSpec((F, F), lambda i: (0, 0)),
            pl.BlockSpec((1, F), lambda i: (0, 0)),
            pl.BlockSpec((F, F), lambda i: (0, 0)),
            pl.BlockSpec((1, F), lambda i: (0, 0)),
        ],
        out_specs=[
            pl.BlockSpec((BM, F), lambda i: (i, 0)),
            pl.BlockSpec((1, F), lambda i: (0, 0)),
            pl.BlockSpec((1, F), lambda i: (0, 0)),
        ],
        out_shape=[
            jax.ShapeDtypeStruct((N, F), jnp.float32),
            jax.ShapeDtypeStruct((1, F), jnp.float32),
            jax.ShapeDtypeStruct((1, F), jnp.float32),
        ],
    )(h, agg[0], agg[1], p["W1"], p["b1"].reshape(1, F), p["W2"],
      p["b2"].reshape(1, F))


def _norm_body(t_ref, s1_ref, s2_ref, g_ref, be_ref, o_ref):
    mean = s1_ref[...]
    var = s2_ref[...] / N
    a = g_ref[...] * lax.rsqrt(var + BN_EPS)
    c = be_ref[...] - mean * a
    o_ref[...] = t_ref[...] * a + c


def _norm(t, s1, s2, gamma, beta):
    return pl.pallas_call(
        _norm_body,
        grid=(NB,),
        in_specs=[
            pl.BlockSpec((BM, F), lambda i: (i, 0)),
            pl.BlockSpec((1, F), lambda i: (0, 0)),
            pl.BlockSpec((1, F), lambda i: (0, 0)),
            pl.BlockSpec((1, F), lambda i: (0, 0)),
            pl.BlockSpec((1, F), lambda i: (0, 0)),
        ],
        out_specs=pl.BlockSpec((BM, F), lambda i: (i, 0)),
        out_shape=jax.ShapeDtypeStruct((N, F), jnp.float32),
    )(t, s1, s2, gamma.reshape(1, F), beta.reshape(1, F))


def _mlp2_body(h_ref, a0_ref, a1_ref,
               w1m_ref, b1m_ref, w2m_ref, b2m_ref,
               w1v_ref, b1v_ref, w2v_ref, b2v_ref,
               tm_ref, s1m_ref, s2m_ref, tv_ref, s1v_ref, s2v_ref):
    z = h_ref[...] + a0_ref[...] + a1_ref[...]
    i = pl.program_id(0)
    for (w1, b1, w2, b2, t_ref, s1_ref, s2_ref) in (
            (w1m_ref, b1m_ref, w2m_ref, b2m_ref, tm_ref, s1m_ref, s2m_ref),
            (w1v_ref, b1v_ref, w2v_ref, b2v_ref, tv_ref, s1v_ref, s2v_ref)):
        u = jnp.maximum(
            jnp.dot(z, w1[...], preferred_element_type=jnp.float32) + b1[...],
            0.0)
        t = jnp.maximum(
            jnp.dot(u, w2[...], preferred_element_type=jnp.float32) + b2[...],
            0.0)
        t_ref[...] = t
        mb = jnp.mean(t, axis=0, keepdims=True)
        m2b = jnp.sum((t - mb) * (t - mb), axis=0, keepdims=True)

        @pl.when(i == 0)
        def _():
            s1_ref[...] = mb
            s2_ref[...] = m2b

        @pl.when(i > 0)
        def _():
            na = i.astype(jnp.float32) * BM
            n = na + BM
            delta = mb - s1_ref[...]
            s1_ref[...] += delta * (BM / n)
            s2_ref[...] += m2b + delta * delta * (na * BM / n)


def _mlp2(h, agg, pm, pv):
    row = pl.BlockSpec((BM, F), lambda i: (i, 0))
    mat = pl.BlockSpec((F, F), lambda i: (0, 0))
    vec = pl.BlockSpec((1, F), lambda i: (0, 0))
    return pl.pallas_call(
        _mlp2_body,
        grid=(NB,),
        in_specs=[row, row, row, mat, vec, mat, vec, mat, vec, mat, vec],
        out_specs=[row, vec, vec, row, vec, vec],
        out_shape=[
            jax.ShapeDtypeStruct((N, F), jnp.float32),
            jax.ShapeDtypeStruct((1, F), jnp.float32),
            jax.ShapeDtypeStruct((1, F), jnp.float32),
            jax.ShapeDtypeStruct((N, F), jnp.float32),
            jax.ShapeDtypeStruct((1, F), jnp.float32),
            jax.ShapeDtypeStruct((1, F), jnp.float32),
        ],
    )(h, agg[0], agg[1],
      pm["W1"], pm["b1"].reshape(1, F), pm["W2"], pm["b2"].reshape(1, F),
      pv["W1"], pv["b1"].reshape(1, F), pv["W2"], pv["b2"].reshape(1, F))


def _norm2_body(tm_ref, s1m_ref, s2m_ref, gm_ref, bm_ref,
                tv_ref, s1v_ref, s2v_ref, gv_ref, bv_ref,
                om_ref, ov_ref):
    for (t_ref, s1_ref, s2_ref, g_ref, be_ref, o_ref) in (
            (tm_ref, s1m_ref, s2m_ref, gm_ref, bm_ref, om_ref),
            (tv_ref, s1v_ref, s2v_ref, gv_ref, bv_ref, ov_ref)):
        mean = s1_ref[...]
        var = s2_ref[...] / N
        a = g_ref[...] * lax.rsqrt(var + BN_EPS)
        c = be_ref[...] - mean * a
        o_ref[...] = t_ref[...] * a + c


def _norm2(tm, s1m, s2m, pm, tv, s1v, s2v, pv):
    row = pl.BlockSpec((BM, F), lambda i: (i, 0))
    vec = pl.BlockSpec((1, F), lambda i: (0, 0))
    return pl.pallas_call(
        _norm2_body,
        grid=(NB,),
        in_specs=[row, vec, vec, vec, vec, row, vec, vec, vec, vec],
        out_specs=[row, row],
        out_shape=[
            jax.ShapeDtypeStruct((N, F), jnp.float32),
            jax.ShapeDtypeStruct((N, F), jnp.float32),
        ],
    )(tm, s1m, s2m, pm["gamma"].reshape(1, F), pm["beta"].reshape(1, F),
      tv, s1v, s2v, pv["gamma"].reshape(1, F), pv["beta"].reshape(1, F))


def _pool_body(h_ref, bat_ref, w1m_ref, b1m_ref, w2m_ref, b2m_ref,
               w1v_ref, b1v_ref, w2v_ref, b2v_ref,
               mu_ref, lv_ref, pooled, cnt):
    i = pl.program_id(0)
    bat = bat_ref[...].reshape(1, PB)
    seg = lax.broadcasted_iota(jnp.int32, (G, PB), 0)
    onehot = jnp.where(bat == seg, 1.0, 0.0)
    pt = lax.dot_general(onehot, h_ref[...], (((1,), (0,)), ((), ())),
                         preferred_element_type=jnp.float32)
    pc = jnp.sum(onehot, axis=1, keepdims=True)

    @pl.when(i == 0)
    def _():
        pooled[...] = jnp.zeros_like(pooled)
        cnt[...] = jnp.zeros_like(cnt)

    pooled[...] += pt
    cnt[...] += pc

    @pl.when(i == NPB - 1)
    def _():
        g = pooled[...] * (1.0 / jnp.maximum(cnt[...], 1.0))
        um = jnp.maximum(
            jnp.dot(g, w1m_ref[...], preferred_element_type=jnp.float32)
            + b1m_ref[...], 0.0)
        mu_ref[...] = jnp.maximum(
            jnp.dot(um, w2m_ref[...], preferred_element_type=jnp.float32)
            + b2m_ref[...], 0.0)
        uv = jnp.maximum(
            jnp.dot(g, w1v_ref[...], preferred_element_type=jnp.float32)
            + b1v_ref[...], 0.0)
        lv_ref[...] = jnp.maximum(
            jnp.dot(uv, w2v_ref[...], preferred_element_type=jnp.float32)
            + b2v_ref[...], 0.0)


def _pool_classify(h, batf, pm, pv):
    return pl.pallas_call(
        _pool_body,
        grid=(NPB,),
        in_specs=[
            pl.BlockSpec((PB, F), lambda i: (i, 0)),
            pl.BlockSpec((1, 1, PB), lambda i: (i, 0, 0)),
            pl.BlockSpec((F, F), lambda i: (0, 0)),
            pl.BlockSpec((1, F), lambda i: (0, 0)),
            pl.BlockSpec((F, F), lambda i: (0, 0)),
            pl.BlockSpec((1, F), lambda i: (0, 0)),
            pl.BlockSpec((F, F), lambda i: (0, 0)),
            pl.BlockSpec((1, F), lambda i: (0, 0)),
            pl.BlockSpec((F, F), lambda i: (0, 0)),
            pl.BlockSpec((1, F), lambda i: (0, 0)),
        ],
        out_specs=[
            pl.BlockSpec((G, F), lambda i: (0, 0)),
            pl.BlockSpec((G, F), lambda i: (0, 0)),
        ],
        out_shape=[
            jax.ShapeDtypeStruct((G, F), jnp.float32),
            jax.ShapeDtypeStruct((G, F), jnp.float32),
        ],
        scratch_shapes=[
            pltpu.VMEM((G, F), jnp.float32),
            pltpu.VMEM((G, 1), jnp.float32),
        ],
    )(h, batf,
      pm["W1"], pm["b1"].reshape(1, F), pm["W2"], pm["b2"].reshape(1, F),
      pv["W1"], pv["b1"].reshape(1, F), pv["W2"], pv["b2"].reshape(1, F))


def kernel(x, edge_index, batch, params):
    src = edge_index[0].astype(jnp.int32).reshape(NW, CHUNKS, K)
    dst = edge_index[1].astype(jnp.int32).reshape(NW, CHUNKS, K)
    batf = batch.astype(jnp.int32).reshape(NPB, 1, PB)

    h = x
    for i in range(3):
        p = params["convs"][i]
        agg = _sc_agg(h, src, dst)
        t, s1, s2 = _mlp(h, agg, p)
        h = _norm(t, s1, s2, p["gamma"], p["beta"])

    class_mu, class_logv = _pool_classify(h, batf, params["cls_mu"],
                                          params["cls_logv"])

    agg = _sc_agg(h, src, dst)
    p3 = params["convs"][3]
    p4 = params["convs"][4]
    t_mu, s1m, s2m, t_lv, s1v, s2v = _mlp2(h, agg, p3, p4)
    node_mu, node_logv = _norm2(t_mu, s1m, s2m, p3, t_lv, s1v, s2v, p4)
    return (node_mu, node_logv, class_mu, class_logv)


# speedup vs baseline: 9.5323x; 1.0510x over previous
import functools

import jax
import jax.numpy as jnp
from jax import lax
from jax.experimental import pallas as pl
from jax.experimental.pallas import tpu as pltpu
from jax.experimental.pallas import tpu_sc as plsc

N = 10000
E = 320000
F = 128
G = 128
BN_EPS = 1e-5

NC = 2
NS = 16
NW = NC * NS

K = 125
CHUNKS = E // (NW * K)
HC = CHUNKS // 2
RPT = 624
TAIL = N - RPT * NS
ZCH = 104

BM = 400
NB = N // BM


def _sc_agg_body(h_hbm, src_hbm, dst_hbm, out_hbm, src_v, dst_v, rows_v, rows_b,
                 acc_sh, sem0, sem1):
    cid = lax.axis_index("c")
    sid = lax.axis_index("s")
    wid = sid * NC + cid

    def _zero_row(i, _):
        for j in range(F // 16):
            rows_v[i, pl.ds(j * 16, 16)] = jnp.zeros((16,), jnp.float32)
        return 0
    lax.fori_loop(0, ZCH, _zero_row, 0)
    for r in range(RPT // ZCH):
        pltpu.sync_copy(rows_v.at[pl.ds(0, ZCH)],
                        acc_sh.at[pl.ds(sid * RPT + r * ZCH, ZCH)])

    @pl.when(sid == NS - 1)
    def _():
        pltpu.sync_copy(rows_v.at[pl.ds(0, TAIL)],
                        acc_sh.at[pl.ds(N - TAIL, TAIL)])

    plsc.subcore_barrier()

    rows = (rows_v, rows_b)
    sems = (sem0, sem1)
    for half in range(2):
        pltpu.sync_copy(src_hbm.at[wid, pl.ds(half * HC, HC)], src_v)
        pltpu.sync_copy(dst_hbm.at[wid, pl.ds(half * HC, HC)], dst_v)
        pltpu.async_copy(h_hbm.at[src_v.at[0]], rows[0], sems[0])
        pltpu.async_copy(h_hbm.at[src_v.at[1]], rows[1], sems[1])

        def _pair(jj, _):
            j = 2 * jj
            for b in range(2):
                pltpu.make_async_copy(h_hbm.at[src_v.at[0]], rows[b],
                                      sems[b]).wait()
                pltpu.sync_copy(rows[b], acc_sh.at[dst_v.at[j + b]], add=True)

                @pl.when(j + b + 2 < HC)
                def _():
                    pltpu.async_copy(h_hbm.at[src_v.at[j + b + 2]], rows[b],
                                     sems[b])
            return 0
        lax.fori_loop(0, HC // 2, _pair, 0)

    plsc.subcore_barrier()
    pltpu.sync_copy(acc_sh.at[pl.ds(sid * RPT, RPT)],
                    out_hbm.at[cid, pl.ds(sid * RPT, RPT)])

    @pl.when(sid == NS - 1)
    def _():
        pltpu.sync_copy(acc_sh.at[pl.ds(N - TAIL, TAIL)],
                        out_hbm.at[cid, pl.ds(N - TAIL, TAIL)])


@functools.lru_cache(maxsize=None)
def _get_sc_agg():
    return pl.kernel(
        _sc_agg_body,
        out_type=jax.ShapeDtypeStruct((NC, N, F), jnp.float32),
        mesh=plsc.VectorSubcoreMesh(core_axis_name="c", subcore_axis_name="s"),
        scratch_types=[
            pltpu.VMEM((HC, K), jnp.int32),
            pltpu.VMEM((HC, K), jnp.int32),
            pltpu.VMEM((K, F), jnp.float32),
            pltpu.VMEM((K, F), jnp.float32),
            pltpu.VMEM_SHARED((N, F), jnp.float32),
            pltpu.SemaphoreType.DMA,
            pltpu.SemaphoreType.DMA,
        ],
    )


def _sc_agg(h, src, dst):
    return _get_sc_agg()(h, src, dst)


def _gin_layer_body(h_ref, a0_ref, a1_ref, w1_ref, b1_ref, w2_ref, b2_ref,
                    g_ref, be_ref, o_ref, t_sc, s1_sc, s2_sc):
    ph = pl.program_id(0)
    i = pl.program_id(1)

    @pl.when(ph == 0)
    def _():
        z = h_ref[...] + a0_ref[...] + a1_ref[...]
        u = jnp.maximum(
            jnp.dot(z, w1_ref[...], preferred_element_type=jnp.float32)
            + b1_ref[...], 0.0)
        t = jnp.maximum(
            jnp.dot(u, w2_ref[...], preferred_element_type=jnp.float32)
            + b2_ref[...], 0.0)
        t_sc[pl.ds(i * BM, BM), :] = t
        mb = jnp.mean(t, axis=0, keepdims=True)
        m2b = jnp.sum((t - mb) * (t - mb), axis=0, keepdims=True)

        @pl.when(i == 0)
        def _():
            s1_sc[...] = mb
            s2_sc[...] = m2b

        @pl.when(i > 0)
        def _():
            na = i.astype(jnp.float32) * BM
            n = na + BM
            delta = mb - s1_sc[...]
            s1_sc[...] += delta * (BM / n)
            s2_sc[...] += m2b + delta * delta * (na * BM / n)

    @pl.when(ph == 1)
    def _():
        a = g_ref[...] * lax.rsqrt(s2_sc[...] / N + BN_EPS)
        c = be_ref[...] - s1_sc[...] * a
        o_ref[...] = t_sc[pl.ds(i * BM, BM), :] * a + c


def _gin_layer(h, agg, p):
    rowin = pl.BlockSpec((BM, F), lambda ph, i: (i * (1 - ph), 0))
    mat = pl.BlockSpec((F, F), lambda ph, i: (0, 0))
    vec = pl.BlockSpec((1, F), lambda ph, i: (0, 0))
    return pl.pallas_call(
        _gin_layer_body,
        grid=(2, NB),
        in_specs=[rowin, rowin, rowin, mat, vec, mat, vec, vec, vec],
        out_specs=pl.BlockSpec((BM, F), lambda ph, i: (i * ph, 0)),
        out_shape=jax.ShapeDtypeStruct((N, F), jnp.float32),
        scratch_shapes=[
            pltpu.VMEM((N, F), jnp.float32),
            pltpu.VMEM((1, F), jnp.float32),
            pltpu.VMEM((1, F), jnp.float32),
        ],
    )(h, agg[0], agg[1], p["W1"], p["b1"].reshape(1, F), p["W2"],
      p["b2"].reshape(1, F), p["gamma"].reshape(1, F), p["beta"].reshape(1, F))


def _gin_heads_body(h_ref, a0_ref, a1_ref, bat_ref,
                    w1m_ref, b1m_ref, w2m_ref, b2m_ref, gm_ref, bm_ref,
                    w1v_ref, b1v_ref, w2v_ref, b2v_ref, gv_ref, bv_ref,
                    cw1m_ref, cb1m_ref, cw2m_ref, cb2m_ref,
                    cw1v_ref, cb1v_ref, cw2v_ref, cb2v_ref,
                    om_ref, ov_ref, cm_ref, cv_ref,
                    tm_sc, tv_sc, s1m_sc, s2m_sc, s1v_sc, s2v_sc,
                    pooled_sc, cnt_sc):
    ph = pl.program_id(0)
    i = pl.program_id(1)

    @pl.when(ph == 0)
    def _():
        h = h_ref[...]
        z = h + a0_ref[...] + a1_ref[...]
        for (w1, b1, w2, b2, t_sc, s1_sc, s2_sc) in (
                (w1m_ref, b1m_ref, w2m_ref, b2m_ref, tm_sc, s1m_sc, s2m_sc),
                (w1v_ref, b1v_ref, w2v_ref, b2v_ref, tv_sc, s1v_sc, s2v_sc)):
            u = jnp.maximum(
                jnp.dot(z, w1[...], preferred_element_type=jnp.float32)
                + b1[...], 0.0)
            t = jnp.maximum(
                jnp.dot(u, w2[...], preferred_element_type=jnp.float32)
                + b2[...], 0.0)
            t_sc[pl.ds(i * BM, BM), :] = t
            mb = jnp.mean(t, axis=0, keepdims=True)
            m2b = jnp.sum((t - mb) * (t - mb), axis=0, keepdims=True)

            @pl.when(i == 0)
            def _():
                s1_sc[...] = mb
                s2_sc[...] = m2b

            @pl.when(i > 0)
            def _():
                na = i.astype(jnp.float32) * BM
                n = na + BM
                delta = mb - s1_sc[...]
                s1_sc[...] += delta * (BM / n)
                s2_sc[...] += m2b + delta * delta * (na * BM / n)

        bat = bat_ref[...].reshape(1, BM)
        seg = lax.broadcasted_iota(jnp.int32, (G, BM), 0)
        onehot = jnp.where(bat == seg, 1.0, 0.0)
        pt = lax.dot_general(onehot, h, (((1,), (0,)), ((), ())),
                             preferred_element_type=jnp.float32)
        pc = jnp.sum(onehot, axis=1, keepdims=True)

        @pl.when(i == 0)
        def _():
            pooled_sc[...] = pt
            cnt_sc[...] = pc

        @pl.when(i > 0)
        def _():
            pooled_sc[...] += pt
            cnt_sc[...] += pc

        @pl.when(i == NB - 1)
        def _():
            g = pooled_sc[...] * (1.0 / jnp.maximum(cnt_sc[...], 1.0))
            um = jnp.maximum(
                jnp.dot(g, cw1m_ref[...], preferred_element_type=jnp.float32)
                + cb1m_ref[...], 0.0)
            cm_ref[...] = jnp.maximum(
                jnp.dot(um, cw2m_ref[...], preferred_element_type=jnp.float32)
                + cb2m_ref[...], 0.0)
            uv = jnp.maximum(
                jnp.dot(g, cw1v_ref[...], preferred_element_type=jnp.float32)
                + cb1v_ref[...], 0.0)
            cv_ref[...] = jnp.maximum(
                jnp.dot(uv, cw2v_ref[...], preferred_element_type=jnp.float32)
                + cb2v_ref[...], 0.0)

    @pl.when(ph == 1)
    def _():
        for (g_ref, be_ref, t_sc, s1_sc, s2_sc, o_ref) in (
                (gm_ref, bm_ref, tm_sc, s1m_sc, s2m_sc, om_ref),
                (gv_ref, bv_ref, tv_sc, s1v_sc, s2v_sc, ov_ref)):
            a = g_ref[...] * lax.rsqrt(s2_sc[...] / N + BN_EPS)
            c = be_ref[...] - s1_sc[...] * a
            o_ref[...] = t_sc[pl.ds(i * BM, BM), :] * a + c


def _gin_heads(h, agg, batf, pm, pv, cmu, cva):
    rowin = pl.BlockSpec((BM, F), lambda ph, i: (i * (1 - ph), 0))
    rowout = pl.BlockSpec((BM, F), lambda ph, i: (i * ph, 0))
    batspec = pl.BlockSpec((1, 1, BM), lambda ph, i: (i * (1 - ph), 0, 0))
    mat = pl.BlockSpec((F, F), lambda ph, i: (0, 0))
    vec = pl.BlockSpec((1, F), lambda ph, i: (0, 0))
    gout = pl.BlockSpec((G, F), lambda ph, i: (0, 0))
    return pl.pallas_call(
        _gin_heads_body,
        grid=(2, NB),
        in_specs=[rowin, rowin, rowin, batspec,
                  mat, vec, mat, vec, vec, vec,
                  mat, vec, mat, vec, vec, vec,
                  mat, vec, mat, vec,
                  mat, vec, mat, vec],
        out_specs=[rowout, rowout, gout, gout],
        out_shape=[
            jax.ShapeDtypeStruct((N, F), jnp.float32),
            jax.ShapeDtypeStruct((N, F), jnp.float32),
            jax.ShapeDtypeStruct((G, F), jnp.float32),
            jax.ShapeDtypeStruct((G, F), jnp.float32),
        ],
        scratch_shapes=[
            pltpu.VMEM((N, F), jnp.float32),
            pltpu.VMEM((N, F), jnp.float32),
            pltpu.VMEM((1, F), jnp.float32),
            pltpu.VMEM((1, F), jnp.float32),
            pltpu.VMEM((1, F), jnp.float32),
            pltpu.VMEM((1, F), jnp.float32),
            pltpu.VMEM((G, F), jnp.float32),
            pltpu.VMEM((G, 1), jnp.float32),
        ],
    )(h, agg[0], agg[1], batf,
      pm["W1"], pm["b1"].reshape(1, F), pm["W2"], pm["b2"].reshape(1, F),
      pm["gamma"].reshape(1, F), pm["beta"].reshape(1, F),
      pv["W1"], pv["b1"].reshape(1, F), pv["W2"], pv["b2"].reshape(1, F),
      pv["gamma"].reshape(1, F), pv["beta"].reshape(1, F),
      cmu["W1"], cmu["b1"].reshape(1, F), cmu["W2"], cmu["b2"].reshape(1, F),
      cva["W1"], cva["b1"].reshape(1, F), cva["W2"], cva["b2"].reshape(1, F))


def kernel(x, edge_index, batch, params):
    src = edge_index[0].astype(jnp.int32).reshape(NW, CHUNKS, K)
    dst = edge_index[1].astype(jnp.int32).reshape(NW, CHUNKS, K)
    batf = batch.astype(jnp.int32).reshape(NB, 1, BM)

    h = x
    for i in range(3):
        agg = _sc_agg(h, src, dst)
        h = _gin_layer(h, agg, params["convs"][i])

    agg = _sc_agg(h, src, dst)
    node_mu, node_logv, class_mu, class_logv = _gin_heads(
        h, agg, batf, params["convs"][3], params["convs"][4],
        params["cls_mu"], params["cls_logv"])
    return (node_mu, node_logv, class_mu, class_logv)


# speedup vs baseline: 10.5400x; 1.1057x over previous
import functools

import jax
import jax.numpy as jnp
from jax import lax
from jax.experimental import pallas as pl
from jax.experimental.pallas import tpu as pltpu
from jax.experimental.pallas import tpu_sc as plsc

N = 10000
E = 320000
F = 128
G = 128
BN_EPS = 1e-5

NC = 2
NS = 16
NW = NC * NS

K = 125
CHUNKS = E // (NW * K)
HC = CHUNKS // 2
RPT = 624
TAIL = N - RPT * NS
ZCH = 104

BM = 1000
NB = N // BM
PB = 200
NPB = N // PB


def _sc_agg_body(h_hbm, src_hbm, dst_hbm, out_hbm, src_v, dst_v, rows_v, rows_b,
                 acc_sh, sem0, sem1):
    cid = lax.axis_index("c")
    sid = lax.axis_index("s")
    wid = sid * NC + cid

    def _zero_row(i, _):
        for j in range(F // 16):
            rows_v[i, pl.ds(j * 16, 16)] = jnp.zeros((16,), jnp.float32)
        return 0
    lax.fori_loop(0, ZCH, _zero_row, 0)
    for r in range(RPT // ZCH):
        pltpu.sync_copy(rows_v.at[pl.ds(0, ZCH)],
                        acc_sh.at[pl.ds(sid * RPT + r * ZCH, ZCH)])

    @pl.when(sid == NS - 1)
    def _():
        pltpu.sync_copy(rows_v.at[pl.ds(0, TAIL)],
                        acc_sh.at[pl.ds(N - TAIL, TAIL)])

    plsc.subcore_barrier()

    rows = (rows_v, rows_b)
    sems = (sem0, sem1)
    for half in range(2):
        pltpu.sync_copy(src_hbm.at[wid, pl.ds(half * HC, HC)], src_v)
        pltpu.sync_copy(dst_hbm.at[wid, pl.ds(half * HC, HC)], dst_v)
        pltpu.async_copy(h_hbm.at[src_v.at[0]], rows[0], sems[0])
        pltpu.async_copy(h_hbm.at[src_v.at[1]], rows[1], sems[1])

        def _pair(jj, _):
            j = 2 * jj
            for b in range(2):
                pltpu.make_async_copy(h_hbm.at[src_v.at[0]], rows[b],
                                      sems[b]).wait()
                pltpu.sync_copy(rows[b], acc_sh.at[dst_v.at[j + b]], add=True)

                @pl.when(j + b + 2 < HC)
                def _():
                    pltpu.async_copy(h_hbm.at[src_v.at[j + b + 2]], rows[b],
                                     sems[b])
            return 0
        lax.fori_loop(0, HC // 2, _pair, 0)

    plsc.subcore_barrier()
    pltpu.sync_copy(acc_sh.at[pl.ds(sid * RPT, RPT)],
                    out_hbm.at[cid, pl.ds(sid * RPT, RPT)])

    @pl.when(sid == NS - 1)
    def _():
        pltpu.sync_copy(acc_sh.at[pl.ds(N - TAIL, TAIL)],
                        out_hbm.at[cid, pl.ds(N - TAIL, TAIL)])


@functools.lru_cache(maxsize=None)
def _get_sc_agg():
    return pl.kernel(
        _sc_agg_body,
        out_type=jax.ShapeDtypeStruct((NC, N, F), jnp.float32),
        mesh=plsc.VectorSubcoreMesh(core_axis_name="c", subcore_axis_name="s"),
        scratch_types=[
            pltpu.VMEM((HC, K), jnp.int32),
            pltpu.VMEM((HC, K), jnp.int32),
            pltpu.VMEM((K, F), jnp.float32),
            pltpu.VMEM((K, F), jnp.float32),
            pltpu.VMEM_SHARED((N, F), jnp.float32),
            pltpu.SemaphoreType.DMA,
            pltpu.SemaphoreType.DMA,
        ],
    )


def _sc_agg(h, src, dst):
    return _get_sc_agg()(h, src, dst)


def _gin_layer_body(h_ref, a0_ref, a1_ref, w1_ref, b1_ref, w2_ref, b2_ref,
                    g_ref, be_ref, o_ref, t_sc, s1_sc, s2_sc):
    ph = pl.program_id(0)
    i = pl.program_id(1)

    @pl.when(ph == 0)
    def _():
        z = h_ref[...] + a0_ref[...] + a1_ref[...]
        u = jnp.maximum(
            jnp.dot(z, w1_ref[...], preferred_element_type=jnp.float32)
            + b1_ref[...], 0.0)
        t = jnp.maximum(
            jnp.dot(u, w2_ref[...], preferred_element_type=jnp.float32)
            + b2_ref[...], 0.0)
        t_sc[pl.ds(i * BM, BM), :] = t
        mb = jnp.mean(t, axis=0, keepdims=True)
        m2b = jnp.sum((t - mb) * (t - mb), axis=0, keepdims=True)

        @pl.when(i == 0)
        def _():
            s1_sc[...] = mb
            s2_sc[...] = m2b

        @pl.when(i > 0)
        def _():
            na = i.astype(jnp.float32) * BM
            n = na + BM
            delta = mb - s1_sc[...]
            s1_sc[...] += delta * (BM / n)
            s2_sc[...] += m2b + delta * delta * (na * BM / n)

    @pl.when(ph == 1)
    def _():
        a = g_ref[...] * lax.rsqrt(s2_sc[...] / N + BN_EPS)
        c = be_ref[...] - s1_sc[...] * a
        o_ref[...] = t_sc[pl.ds(i * BM, BM), :] * a + c


def _gin_layer(h, agg, p):
    rowin = pl.BlockSpec((BM, F), lambda ph, i: (i * (1 - ph), 0))
    mat = pl.BlockSpec((F, F), lambda ph, i: (0, 0))
    vec = pl.BlockSpec((1, F), lambda ph, i: (0, 0))
    return pl.pallas_call(
        _gin_layer_body,
        grid=(2, NB),
        in_specs=[rowin, rowin, rowin, mat, vec, mat, vec, vec, vec],
        out_specs=pl.BlockSpec((BM, F), lambda ph, i: (i * ph, 0)),
        out_shape=jax.ShapeDtypeStruct((N, F), jnp.float32),
        scratch_shapes=[
            pltpu.VMEM((N, F), jnp.float32),
            pltpu.VMEM((1, F), jnp.float32),
            pltpu.VMEM((1, F), jnp.float32),
        ],
    )(h, agg[0], agg[1], p["W1"], p["b1"].reshape(1, F), p["W2"],
      p["b2"].reshape(1, F), p["gamma"].reshape(1, F), p["beta"].reshape(1, F))


def _gin_heads_body(h_ref, a0_ref, a1_ref,
                    w1m_ref, b1m_ref, w2m_ref, b2m_ref, gm_ref, bm_ref,
                    w1v_ref, b1v_ref, w2v_ref, b2v_ref, gv_ref, bv_ref,
                    om_ref, ov_ref,
                    tm_sc, tv_sc, s1m_sc, s2m_sc, s1v_sc, s2v_sc):
    ph = pl.program_id(0)
    i = pl.program_id(1)

    @pl.when(ph == 0)
    def _():
        z = h_ref[...] + a0_ref[...] + a1_ref[...]
        for (w1, b1, w2, b2, t_sc, s1_sc, s2_sc) in (
                (w1m_ref, b1m_ref, w2m_ref, b2m_ref, tm_sc, s1m_sc, s2m_sc),
                (w1v_ref, b1v_ref, w2v_ref, b2v_ref, tv_sc, s1v_sc, s2v_sc)):
            u = jnp.maximum(
                jnp.dot(z, w1[...], preferred_element_type=jnp.float32)
                + b1[...], 0.0)
            t = jnp.maximum(
                jnp.dot(u, w2[...], preferred_element_type=jnp.float32)
                + b2[...], 0.0)
            t_sc[pl.ds(i * BM, BM), :] = t
            mb = jnp.mean(t, axis=0, keepdims=True)
            m2b = jnp.sum((t - mb) * (t - mb), axis=0, keepdims=True)

            @pl.when(i == 0)
            def _():
                s1_sc[...] = mb
                s2_sc[...] = m2b

            @pl.when(i > 0)
            def _():
                na = i.astype(jnp.float32) * BM
                n = na + BM
                delta = mb - s1_sc[...]
                s1_sc[...] += delta * (BM / n)
                s2_sc[...] += m2b + delta * delta * (na * BM / n)

    @pl.when(ph == 1)
    def _():
        for (g_ref, be_ref, t_sc, s1_sc, s2_sc, o_ref) in (
                (gm_ref, bm_ref, tm_sc, s1m_sc, s2m_sc, om_ref),
                (gv_ref, bv_ref, tv_sc, s1v_sc, s2v_sc, ov_ref)):
            a = g_ref[...] * lax.rsqrt(s2_sc[...] / N + BN_EPS)
            c = be_ref[...] - s1_sc[...] * a
            o_ref[...] = t_sc[pl.ds(i * BM, BM), :] * a + c


def _gin_heads(h, agg, pm, pv):
    rowin = pl.BlockSpec((BM, F), lambda ph, i: (i * (1 - ph), 0))
    rowout = pl.BlockSpec((BM, F), lambda ph, i: (i * ph, 0))
    mat = pl.BlockSpec((F, F), lambda ph, i: (0, 0))
    vec = pl.BlockSpec((1, F), lambda ph, i: (0, 0))
    return pl.pallas_call(
        _gin_heads_body,
        grid=(2, NB),
        in_specs=[rowin, rowin, rowin,
                  mat, vec, mat, vec, vec, vec,
                  mat, vec, mat, vec, vec, vec],
        out_specs=[rowout, rowout],
        out_shape=[
            jax.ShapeDtypeStruct((N, F), jnp.float32),
            jax.ShapeDtypeStruct((N, F), jnp.float32),
        ],
        scratch_shapes=[
            pltpu.VMEM((N, F), jnp.float32),
            pltpu.VMEM((N, F), jnp.float32),
            pltpu.VMEM((1, F), jnp.float32),
            pltpu.VMEM((1, F), jnp.float32),
            pltpu.VMEM((1, F), jnp.float32),
            pltpu.VMEM((1, F), jnp.float32),
        ],
    )(h, agg[0], agg[1],
      pm["W1"], pm["b1"].reshape(1, F), pm["W2"], pm["b2"].reshape(1, F),
      pm["gamma"].reshape(1, F), pm["beta"].reshape(1, F),
      pv["W1"], pv["b1"].reshape(1, F), pv["W2"], pv["b2"].reshape(1, F),
      pv["gamma"].reshape(1, F), pv["beta"].reshape(1, F))


def _pool_body(h_ref, bat_ref, w1m_ref, b1m_ref, w2m_ref, b2m_ref,
               w1v_ref, b1v_ref, w2v_ref, b2v_ref,
               mu_ref, lv_ref, pooled, cnt):
    i = pl.program_id(0)
    bat = bat_ref[...].reshape(1, PB)
    seg = lax.broadcasted_iota(jnp.int32, (G, PB), 0)
    onehot = jnp.where(bat == seg, 1.0, 0.0)
    pt = lax.dot_general(onehot, h_ref[...], (((1,), (0,)), ((), ())),
                         preferred_element_type=jnp.float32)
    pc = jnp.sum(onehot, axis=1, keepdims=True)

    @pl.when(i == 0)
    def _():
        pooled[...] = jnp.zeros_like(pooled)
        cnt[...] = jnp.zeros_like(cnt)

    pooled[...] += pt
    cnt[...] += pc

    @pl.when(i == NPB - 1)
    def _():
        g = pooled[...] * (1.0 / jnp.maximum(cnt[...], 1.0))
        um = jnp.maximum(
            jnp.dot(g, w1m_ref[...], preferred_element_type=jnp.float32)
            + b1m_ref[...], 0.0)
        mu_ref[...] = jnp.maximum(
            jnp.dot(um, w2m_ref[...], preferred_element_type=jnp.float32)
            + b2m_ref[...], 0.0)
        uv = jnp.maximum(
            jnp.dot(g, w1v_ref[...], preferred_element_type=jnp.float32)
            + b1v_ref[...], 0.0)
        lv_ref[...] = jnp.maximum(
            jnp.dot(uv, w2v_ref[...], preferred_element_type=jnp.float32)
            + b2v_ref[...], 0.0)


def _pool_classify(h, batf, pm, pv):
    rowspec = pl.BlockSpec((PB, F), lambda i: (i, 0))
    batspec = pl.BlockSpec((1, 1, PB), lambda i: (i, 0, 0))
    mat = pl.BlockSpec((F, F), lambda i: (0, 0))
    vec = pl.BlockSpec((1, F), lambda i: (0, 0))
    gout = pl.BlockSpec((G, F), lambda i: (0, 0))
    return pl.pallas_call(
        _pool_body,
        grid=(NPB,),
        in_specs=[rowspec, batspec,
                  mat, vec, mat, vec, mat, vec, mat, vec],
        out_specs=[gout, gout],
        out_shape=[
            jax.ShapeDtypeStruct((G, F), jnp.float32),
            jax.ShapeDtypeStruct((G, F), jnp.float32),
        ],
        scratch_shapes=[
            pltpu.VMEM((G, F), jnp.float32),
            pltpu.VMEM((G, 1), jnp.float32),
        ],
    )(h, batf,
      pm["W1"], pm["b1"].reshape(1, F), pm["W2"], pm["b2"].reshape(1, F),
      pv["W1"], pv["b1"].reshape(1, F), pv["W2"], pv["b2"].reshape(1, F))


def kernel(x, edge_index, batch, params):
    src = edge_index[0].astype(jnp.int32).reshape(NW, CHUNKS, K)
    dst = edge_index[1].astype(jnp.int32).reshape(NW, CHUNKS, K)
    batf = batch.astype(jnp.int32).reshape(NPB, 1, PB)

    h = x
    for i in range(3):
        agg = _sc_agg(h, src, dst)
        h = _gin_layer(h, agg, params["convs"][i])

    class_mu, class_logv = _pool_classify(h, batf, params["cls_mu"],
                                          params["cls_logv"])

    agg = _sc_agg(h, src, dst)
    node_mu, node_logv = _gin_heads(h, agg, params["convs"][3],
                                    params["convs"][4])
    return (node_mu, node_logv, class_mu, class_logv)


# speedup vs baseline: 10.9271x; 1.0367x over previous
import functools

import jax
import jax.numpy as jnp
from jax import lax
from jax.experimental import pallas as pl
from jax.experimental.pallas import tpu as pltpu
from jax.experimental.pallas import tpu_sc as plsc

N = 10000
E = 320000
F = 128
G = 128
BN_EPS = 1e-5

NC = 2
NS = 16
NW = NC * NS

K = 125
CHUNKS = E // (NW * K)
HC = CHUNKS // 2
RPT = 624
TAIL = N - RPT * NS
ZCH = 104

BM = 2000
NB = N // BM
PB = 200
NPB = N // PB


def _sc_agg_body(h_hbm, src_hbm, dst_hbm, out_hbm, src_v, dst_v, rows_v, rows_b,
                 acc_sh, sem0, sem1):
    cid = lax.axis_index("c")
    sid = lax.axis_index("s")
    wid = sid * NC + cid

    def _zero_row(i, _):
        for j in range(F // 16):
            rows_v[i, pl.ds(j * 16, 16)] = jnp.zeros((16,), jnp.float32)
        return 0
    lax.fori_loop(0, ZCH, _zero_row, 0)
    for r in range(RPT // ZCH):
        pltpu.sync_copy(rows_v.at[pl.ds(0, ZCH)],
                        acc_sh.at[pl.ds(sid * RPT + r * ZCH, ZCH)])

    @pl.when(sid == NS - 1)
    def _():
        pltpu.sync_copy(rows_v.at[pl.ds(0, TAIL)],
                        acc_sh.at[pl.ds(N - TAIL, TAIL)])

    plsc.subcore_barrier()

    rows = (rows_v, rows_b)
    sems = (sem0, sem1)
    for half in range(2):
        pltpu.sync_copy(src_hbm.at[wid, pl.ds(half * HC, HC)], src_v)
        pltpu.sync_copy(dst_hbm.at[wid, pl.ds(half * HC, HC)], dst_v)
        pltpu.async_copy(h_hbm.at[src_v.at[0]], rows[0], sems[0])
        pltpu.async_copy(h_hbm.at[src_v.at[1]], rows[1], sems[1])

        def _pair(jj, _):
            j = 2 * jj
            for b in range(2):
                pltpu.make_async_copy(h_hbm.at[src_v.at[0]], rows[b],
                                      sems[b]).wait()
                pltpu.sync_copy(rows[b], acc_sh.at[dst_v.at[j + b]], add=True)

                @pl.when(j + b + 2 < HC)
                def _():
                    pltpu.async_copy(h_hbm.at[src_v.at[j + b + 2]], rows[b],
                                     sems[b])
            return 0
        lax.fori_loop(0, HC // 2, _pair, 0)

    plsc.subcore_barrier()
    pltpu.sync_copy(acc_sh.at[pl.ds(sid * RPT, RPT)],
                    out_hbm.at[cid, pl.ds(sid * RPT, RPT)])

    @pl.when(sid == NS - 1)
    def _():
        pltpu.sync_copy(acc_sh.at[pl.ds(N - TAIL, TAIL)],
                        out_hbm.at[cid, pl.ds(N - TAIL, TAIL)])


@functools.lru_cache(maxsize=None)
def _get_sc_agg():
    return pl.kernel(
        _sc_agg_body,
        out_type=jax.ShapeDtypeStruct((NC, N, F), jnp.float32),
        mesh=plsc.VectorSubcoreMesh(core_axis_name="c", subcore_axis_name="s"),
        scratch_types=[
            pltpu.VMEM((HC, K), jnp.int32),
            pltpu.VMEM((HC, K), jnp.int32),
            pltpu.VMEM((K, F), jnp.float32),
            pltpu.VMEM((K, F), jnp.float32),
            pltpu.VMEM_SHARED((N, F), jnp.float32),
            pltpu.SemaphoreType.DMA,
            pltpu.SemaphoreType.DMA,
        ],
    )


def _sc_agg(h, src, dst):
    return _get_sc_agg()(h, src, dst)


def _gin_layer_body(h_ref, a0_ref, a1_ref, w1_ref, b1_ref, w2_ref, b2_ref,
                    g_ref, be_ref, o_ref, t_sc, s1_sc, s2_sc):
    ph = pl.program_id(0)
    i = pl.program_id(1)

    @pl.when(ph == 0)
    def _():
        z = h_ref[...] + a0_ref[...] + a1_ref[...]
        u = jnp.maximum(
            jnp.dot(z, w1_ref[...], preferred_element_type=jnp.float32)
            + b1_ref[...], 0.0)
        t = jnp.maximum(
            jnp.dot(u, w2_ref[...], preferred_element_type=jnp.float32)
            + b2_ref[...], 0.0)
        t_sc[pl.ds(i * BM, BM), :] = t
        mb = jnp.mean(t, axis=0, keepdims=True)
        m2b = jnp.sum((t - mb) * (t - mb), axis=0, keepdims=True)

        @pl.when(i == 0)
        def _():
            s1_sc[...] = mb
            s2_sc[...] = m2b

        @pl.when(i > 0)
        def _():
            na = i.astype(jnp.float32) * BM
            n = na + BM
            delta = mb - s1_sc[...]
            s1_sc[...] += delta * (BM / n)
            s2_sc[...] += m2b + delta * delta * (na * BM / n)

    @pl.when(ph == 1)
    def _():
        a = g_ref[...] * lax.rsqrt(s2_sc[...] / N + BN_EPS)
        c = be_ref[...] - s1_sc[...] * a
        o_ref[...] = t_sc[pl.ds(i * BM, BM), :] * a + c


def _gin_layer(h, agg, p):
    rowin = pl.BlockSpec((BM, F), lambda ph, i: (i * (1 - ph), 0))
    mat = pl.BlockSpec((F, F), lambda ph, i: (0, 0))
    vec = pl.BlockSpec((1, F), lambda ph, i: (0, 0))
    return pl.pallas_call(
        _gin_layer_body,
        grid=(2, NB),
        in_specs=[rowin, rowin, rowin, mat, vec, mat, vec, vec, vec],
        out_specs=pl.BlockSpec((BM, F), lambda ph, i: (i * ph, 0)),
        out_shape=jax.ShapeDtypeStruct((N, F), jnp.float32),
        scratch_shapes=[
            pltpu.VMEM((N, F), jnp.float32),
            pltpu.VMEM((1, F), jnp.float32),
            pltpu.VMEM((1, F), jnp.float32),
        ],
    )(h, agg[0], agg[1], p["W1"], p["b1"].reshape(1, F), p["W2"],
      p["b2"].reshape(1, F), p["gamma"].reshape(1, F), p["beta"].reshape(1, F))


def _gin_heads_body(h_ref, a0_ref, a1_ref,
                    w1m_ref, b1m_ref, w2m_ref, b2m_ref, gm_ref, bm_ref,
                    w1v_ref, b1v_ref, w2v_ref, b2v_ref, gv_ref, bv_ref,
                    om_ref, ov_ref,
                    tm_sc, tv_sc, s1m_sc, s2m_sc, s1v_sc, s2v_sc):
    ph = pl.program_id(0)
    i = pl.program_id(1)

    @pl.when(ph == 0)
    def _():
        z = h_ref[...] + a0_ref[...] + a1_ref[...]
        for (w1, b1, w2, b2, t_sc, s1_sc, s2_sc) in (
                (w1m_ref, b1m_ref, w2m_ref, b2m_ref, tm_sc, s1m_sc, s2m_sc),
                (w1v_ref, b1v_ref, w2v_ref, b2v_ref, tv_sc, s1v_sc, s2v_sc)):
            u = jnp.maximum(
                jnp.dot(z, w1[...], preferred_element_type=jnp.float32)
                + b1[...], 0.0)
            t = jnp.maximum(
                jnp.dot(u, w2[...], preferred_element_type=jnp.float32)
                + b2[...], 0.0)
            t_sc[pl.ds(i * BM, BM), :] = t
            mb = jnp.mean(t, axis=0, keepdims=True)
            m2b = jnp.sum((t - mb) * (t - mb), axis=0, keepdims=True)

            @pl.when(i == 0)
            def _():
                s1_sc[...] = mb
                s2_sc[...] = m2b

            @pl.when(i > 0)
            def _():
                na = i.astype(jnp.float32) * BM
                n = na + BM
                delta = mb - s1_sc[...]
                s1_sc[...] += delta * (BM / n)
                s2_sc[...] += m2b + delta * delta * (na * BM / n)

    @pl.when(ph == 1)
    def _():
        for (g_ref, be_ref, t_sc, s1_sc, s2_sc, o_ref) in (
                (gm_ref, bm_ref, tm_sc, s1m_sc, s2m_sc, om_ref),
                (gv_ref, bv_ref, tv_sc, s1v_sc, s2v_sc, ov_ref)):
            a = g_ref[...] * lax.rsqrt(s2_sc[...] / N + BN_EPS)
            c = be_ref[...] - s1_sc[...] * a
            o_ref[...] = t_sc[pl.ds(i * BM, BM), :] * a + c


def _gin_heads(h, agg, pm, pv):
    rowin = pl.BlockSpec((BM, F), lambda ph, i: (i * (1 - ph), 0))
    rowout = pl.BlockSpec((BM, F), lambda ph, i: (i * ph, 0))
    mat = pl.BlockSpec((F, F), lambda ph, i: (0, 0))
    vec = pl.BlockSpec((1, F), lambda ph, i: (0, 0))
    return pl.pallas_call(
        _gin_heads_body,
        grid=(2, NB),
        in_specs=[rowin, rowin, rowin,
                  mat, vec, mat, vec, vec, vec,
                  mat, vec, mat, vec, vec, vec],
        out_specs=[rowout, rowout],
        out_shape=[
            jax.ShapeDtypeStruct((N, F), jnp.float32),
            jax.ShapeDtypeStruct((N, F), jnp.float32),
        ],
        scratch_shapes=[
            pltpu.VMEM((N, F), jnp.float32),
            pltpu.VMEM((N, F), jnp.float32),
            pltpu.VMEM((1, F), jnp.float32),
            pltpu.VMEM((1, F), jnp.float32),
            pltpu.VMEM((1, F), jnp.float32),
            pltpu.VMEM((1, F), jnp.float32),
        ],
    )(h, agg[0], agg[1],
      pm["W1"], pm["b1"].reshape(1, F), pm["W2"], pm["b2"].reshape(1, F),
      pm["gamma"].reshape(1, F), pm["beta"].reshape(1, F),
      pv["W1"], pv["b1"].reshape(1, F), pv["W2"], pv["b2"].reshape(1, F),
      pv["gamma"].reshape(1, F), pv["beta"].reshape(1, F))


def _pool_body(h_ref, bat_ref, w1m_ref, b1m_ref, w2m_ref, b2m_ref,
               w1v_ref, b1v_ref, w2v_ref, b2v_ref,
               mu_ref, lv_ref, pooled, cnt):
    i = pl.program_id(0)
    bat = bat_ref[...].reshape(1, PB)
    seg = lax.broadcasted_iota(jnp.int32, (G, PB), 0)
    onehot = jnp.where(bat == seg, 1.0, 0.0)
    pt = lax.dot_general(onehot, h_ref[...], (((1,), (0,)), ((), ())),
                         preferred_element_type=jnp.float32)
    pc = jnp.sum(onehot, axis=1, keepdims=True)

    @pl.when(i == 0)
    def _():
        pooled[...] = jnp.zeros_like(pooled)
        cnt[...] = jnp.zeros_like(cnt)

    pooled[...] += pt
    cnt[...] += pc

    @pl.when(i == NPB - 1)
    def _():
        g = pooled[...] * (1.0 / jnp.maximum(cnt[...], 1.0))
        um = jnp.maximum(
            jnp.dot(g, w1m_ref[...], preferred_element_type=jnp.float32)
            + b1m_ref[...], 0.0)
        mu_ref[...] = jnp.maximum(
            jnp.dot(um, w2m_ref[...], preferred_element_type=jnp.float32)
            + b2m_ref[...], 0.0)
        uv = jnp.maximum(
            jnp.dot(g, w1v_ref[...], preferred_element_type=jnp.float32)
            + b1v_ref[...], 0.0)
        lv_ref[...] = jnp.maximum(
            jnp.dot(uv, w2v_ref[...], preferred_element_type=jnp.float32)
            + b2v_ref[...], 0.0)


def _pool_classify(h, batf, pm, pv):
    rowspec = pl.BlockSpec((PB, F), lambda i: (i, 0))
    batspec = pl.BlockSpec((1, 1, PB), lambda i: (i, 0, 0))
    mat = pl.BlockSpec((F, F), lambda i: (0, 0))
    vec = pl.BlockSpec((1, F), lambda i: (0, 0))
    gout = pl.BlockSpec((G, F), lambda i: (0, 0))
    return pl.pallas_call(
        _pool_body,
        grid=(NPB,),
        in_specs=[rowspec, batspec,
                  mat, vec, mat, vec, mat, vec, mat, vec],
        out_specs=[gout, gout],
        out_shape=[
            jax.ShapeDtypeStruct((G, F), jnp.float32),
            jax.ShapeDtypeStruct((G, F), jnp.float32),
        ],
        scratch_shapes=[
            pltpu.VMEM((G, F), jnp.float32),
            pltpu.VMEM((G, 1), jnp.float32),
        ],
    )(h, batf,
      pm["W1"], pm["b1"].reshape(1, F), pm["W2"], pm["b2"].reshape(1, F),
      pv["W1"], pv["b1"].reshape(1, F), pv["W2"], pv["b2"].reshape(1, F))


def kernel(x, edge_index, batch, params):
    src = edge_index[0].astype(jnp.int32).reshape(NW, CHUNKS, K)
    dst = edge_index[1].astype(jnp.int32).reshape(NW, CHUNKS, K)
    batf = batch.astype(jnp.int32).reshape(NPB, 1, PB)

    h = x
    for i in range(3):
        agg = _sc_agg(h, src, dst)
        h = _gin_layer(h, agg, params["convs"][i])

    class_mu, class_logv = _pool_classify(h, batf, params["cls_mu"],
                                          params["cls_logv"])

    agg = _sc_agg(h, src, dst)
    node_mu, node_logv = _gin_heads(h, agg, params["convs"][3],
                                    params["convs"][4])
    return (node_mu, node_logv, class_mu, class_logv)


# speedup vs baseline: 11.1132x; 1.0170x over previous
import functools

import jax
import jax.numpy as jnp
from jax import lax
from jax.experimental import pallas as pl
from jax.experimental.pallas import tpu as pltpu
from jax.experimental.pallas import tpu_sc as plsc

N = 10000
E = 320000
F = 128
G = 128
BN_EPS = 1e-5

NC = 2
NS = 16
NW = NC * NS

K = 125
CHUNKS = E // (NW * K)
HC = CHUNKS // 2
RPT = 624
TAIL = N - RPT * NS
ZCH = 52

BM = 2000
NB = N // BM
PB = 200
NPB = N // PB


def _sc_agg_body(h_hbm, src_hbm, dst_hbm, out_hbm, src_v, dst_v, rows_v, rows_b,
                 zb_v, acc_sh, sem0, sem1, semz):
    cid = lax.axis_index("c")
    sid = lax.axis_index("s")
    wid = sid * NC + cid
    rows = (rows_v, rows_b)
    sems = (sem0, sem1)

    pltpu.sync_copy(src_hbm.at[wid, pl.ds(0, HC)], src_v)
    pltpu.sync_copy(dst_hbm.at[wid, pl.ds(0, HC)], dst_v)
    pltpu.async_copy(h_hbm.at[src_v.at[0]], rows[0], sems[0])
    pltpu.async_copy(h_hbm.at[src_v.at[1]], rows[1], sems[1])

    def _zero_row(i, _):
        for j in range(F // 16):
            zb_v[i, pl.ds(j * 16, 16)] = jnp.zeros((16,), jnp.float32)
        return 0
    lax.fori_loop(0, ZCH, _zero_row, 0)
    for r in range(RPT // ZCH):
        pltpu.async_copy(zb_v.at[pl.ds(0, ZCH)],
                         acc_sh.at[pl.ds(sid * RPT + r * ZCH, ZCH)], semz)

    @pl.when(sid == NS - 1)
    def _():
        pltpu.async_copy(zb_v.at[pl.ds(0, TAIL)],
                         acc_sh.at[pl.ds(N - TAIL, TAIL)], semz)

    for r in range(RPT // ZCH):
        pltpu.make_async_copy(zb_v.at[pl.ds(0, ZCH)],
                              acc_sh.at[pl.ds(sid * RPT, ZCH)], semz).wait()

    @pl.when(sid == NS - 1)
    def _():
        pltpu.make_async_copy(zb_v.at[pl.ds(0, TAIL)],
                              acc_sh.at[pl.ds(N - TAIL, TAIL)], semz).wait()

    plsc.subcore_barrier()

    for half in range(2):
        if half == 1:
            pltpu.sync_copy(src_hbm.at[wid, pl.ds(HC, HC)], src_v)
            pltpu.sync_copy(dst_hbm.at[wid, pl.ds(HC, HC)], dst_v)
            pltpu.async_copy(h_hbm.at[src_v.at[0]], rows[0], sems[0])
            pltpu.async_copy(h_hbm.at[src_v.at[1]], rows[1], sems[1])

        def _pair(jj, _):
            j = 2 * jj
            for b in range(2):
                pltpu.make_async_copy(h_hbm.at[src_v.at[0]], rows[b],
                                      sems[b]).wait()
                pltpu.sync_copy(rows[b], acc_sh.at[dst_v.at[j + b]], add=True)

                @pl.when(j + b + 2 < HC)
                def _():
                    pltpu.async_copy(h_hbm.at[src_v.at[j + b + 2]], rows[b],
                                     sems[b])
            return 0
        lax.fori_loop(0, HC // 2, _pair, 0)

    plsc.subcore_barrier()
    pltpu.sync_copy(acc_sh.at[pl.ds(sid * RPT, RPT)],
                    out_hbm.at[cid, pl.ds(sid * RPT, RPT)])

    @pl.when(sid == NS - 1)
    def _():
        pltpu.sync_copy(acc_sh.at[pl.ds(N - TAIL, TAIL)],
                        out_hbm.at[cid, pl.ds(N - TAIL, TAIL)])


@functools.lru_cache(maxsize=None)
def _get_sc_agg():
    return pl.kernel(
        _sc_agg_body,
        out_type=jax.ShapeDtypeStruct((NC, N, F), jnp.float32),
        mesh=plsc.VectorSubcoreMesh(core_axis_name="c", subcore_axis_name="s"),
        scratch_types=[
            pltpu.VMEM((HC, K), jnp.int32),
            pltpu.VMEM((HC, K), jnp.int32),
            pltpu.VMEM((K, F), jnp.float32),
            pltpu.VMEM((K, F), jnp.float32),
            pltpu.VMEM((ZCH, F), jnp.float32),
            pltpu.VMEM_SHARED((N, F), jnp.float32),
            pltpu.SemaphoreType.DMA,
            pltpu.SemaphoreType.DMA,
            pltpu.SemaphoreType.DMA,
        ],
    )


def _sc_agg(h, src, dst):
    return _get_sc_agg()(h, src, dst)


def _gin_layer_body(h_ref, a0_ref, a1_ref, w1_ref, b1_ref, w2_ref, b2_ref,
                    g_ref, be_ref, o_ref, t_sc, s1_sc, s2_sc):
    ph = pl.program_id(0)
    i = pl.program_id(1)

    @pl.when(ph == 0)
    def _():
        z = h_ref[...] + a0_ref[...] + a1_ref[...]
        u = jnp.maximum(
            jnp.dot(z, w1_ref[...], preferred_element_type=jnp.float32)
            + b1_ref[...], 0.0)
        t = jnp.maximum(
            jnp.dot(u, w2_ref[...], preferred_element_type=jnp.float32)
            + b2_ref[...], 0.0)
        t_sc[pl.ds(i * BM, BM), :] = t
        mb = jnp.mean(t, axis=0, keepdims=True)
        m2b = jnp.sum((t - mb) * (t - mb), axis=0, keepdims=True)

        @pl.when(i == 0)
        def _():
            s1_sc[...] = mb
            s2_sc[...] = m2b

        @pl.when(i > 0)
        def _():
            na = i.astype(jnp.float32) * BM
            n = na + BM
            delta = mb - s1_sc[...]
            s1_sc[...] += delta * (BM / n)
            s2_sc[...] += m2b + delta * delta * (na * BM / n)

    @pl.when(ph == 1)
    def _():
        a = g_ref[...] * lax.rsqrt(s2_sc[...] / N + BN_EPS)
        c = be_ref[...] - s1_sc[...] * a
        o_ref[...] = t_sc[pl.ds(i * BM, BM), :] * a + c


def _gin_layer(h, agg, p):
    rowin = pl.BlockSpec((BM, F), lambda ph, i: (i * (1 - ph), 0))
    mat = pl.BlockSpec((F, F), lambda ph, i: (0, 0))
    vec = pl.BlockSpec((1, F), lambda ph, i: (0, 0))
    return pl.pallas_call(
        _gin_layer_body,
        grid=(2, NB),
        in_specs=[rowin, rowin, rowin, mat, vec, mat, vec, vec, vec],
        out_specs=pl.BlockSpec((BM, F), lambda ph, i: (i * ph, 0)),
        out_shape=jax.ShapeDtypeStruct((N, F), jnp.float32),
        scratch_shapes=[
            pltpu.VMEM((N, F), jnp.float32),
            pltpu.VMEM((1, F), jnp.float32),
            pltpu.VMEM((1, F), jnp.float32),
        ],
    )(h, agg[0], agg[1], p["W1"], p["b1"].reshape(1, F), p["W2"],
      p["b2"].reshape(1, F), p["gamma"].reshape(1, F), p["beta"].reshape(1, F))


def _gin_heads_body(h_ref, a0_ref, a1_ref,
                    w1m_ref, b1m_ref, w2m_ref, b2m_ref, gm_ref, bm_ref,
                    w1v_ref, b1v_ref, w2v_ref, b2v_ref, gv_ref, bv_ref,
                    om_ref, ov_ref,
                    tm_sc, tv_sc, s1m_sc, s2m_sc, s1v_sc, s2v_sc):
    ph = pl.program_id(0)
    i = pl.program_id(1)

    @pl.when(ph == 0)
    def _():
        z = h_ref[...] + a0_ref[...] + a1_ref[...]
        for (w1, b1, w2, b2, t_sc, s1_sc, s2_sc) in (
                (w1m_ref, b1m_ref, w2m_ref, b2m_ref, tm_sc, s1m_sc, s2m_sc),
                (w1v_ref, b1v_ref, w2v_ref, b2v_ref, tv_sc, s1v_sc, s2v_sc)):
            u = jnp.maximum(
                jnp.dot(z, w1[...], preferred_element_type=jnp.float32)
                + b1[...], 0.0)
            t = jnp.maximum(
                jnp.dot(u, w2[...], preferred_element_type=jnp.float32)
                + b2[...], 0.0)
            t_sc[pl.ds(i * BM, BM), :] = t
            mb = jnp.mean(t, axis=0, keepdims=True)
            m2b = jnp.sum((t - mb) * (t - mb), axis=0, keepdims=True)

            @pl.when(i == 0)
            def _():
                s1_sc[...] = mb
                s2_sc[...] = m2b

            @pl.when(i > 0)
            def _():
                na = i.astype(jnp.float32) * BM
                n = na + BM
                delta = mb - s1_sc[...]
                s1_sc[...] += delta * (BM / n)
                s2_sc[...] += m2b + delta * delta * (na * BM / n)

    @pl.when(ph == 1)
    def _():
        for (g_ref, be_ref, t_sc, s1_sc, s2_sc, o_ref) in (
                (gm_ref, bm_ref, tm_sc, s1m_sc, s2m_sc, om_ref),
                (gv_ref, bv_ref, tv_sc, s1v_sc, s2v_sc, ov_ref)):
            a = g_ref[...] * lax.rsqrt(s2_sc[...] / N + BN_EPS)
            c = be_ref[...] - s1_sc[...] * a
            o_ref[...] = t_sc[pl.ds(i * BM, BM), :] * a + c


def _gin_heads(h, agg, pm, pv):
    rowin = pl.BlockSpec((BM, F), lambda ph, i: (i * (1 - ph), 0))
    rowout = pl.BlockSpec((BM, F), lambda ph, i: (i * ph, 0))
    mat = pl.BlockSpec((F, F), lambda ph, i: (0, 0))
    vec = pl.BlockSpec((1, F), lambda ph, i: (0, 0))
    return pl.pallas_call(
        _gin_heads_body,
        grid=(2, NB),
        in_specs=[rowin, rowin, rowin,
                  mat, vec, mat, vec, vec, vec,
                  mat, vec, mat, vec, vec, vec],
        out_specs=[rowout, rowout],
        out_shape=[
            jax.ShapeDtypeStruct((N, F), jnp.float32),
            jax.ShapeDtypeStruct((N, F), jnp.float32),
        ],
        scratch_shapes=[
            pltpu.VMEM((N, F), jnp.float32),
            pltpu.VMEM((N, F), jnp.float32),
            pltpu.VMEM((1, F), jnp.float32),
            pltpu.VMEM((1, F), jnp.float32),
            pltpu.VMEM((1, F), jnp.float32),
            pltpu.VMEM((1, F), jnp.float32),
        ],
    )(h, agg[0], agg[1],
      pm["W1"], pm["b1"].reshape(1, F), pm["W2"], pm["b2"].reshape(1, F),
      pm["gamma"].reshape(1, F), pm["beta"].reshape(1, F),
      pv["W1"], pv["b1"].reshape(1, F), pv["W2"], pv["b2"].reshape(1, F),
      pv["gamma"].reshape(1, F), pv["beta"].reshape(1, F))


def _pool_body(h_ref, bat_ref, w1m_ref, b1m_ref, w2m_ref, b2m_ref,
               w1v_ref, b1v_ref, w2v_ref, b2v_ref,
               mu_ref, lv_ref, pooled, cnt):
    i = pl.program_id(0)
    bat = bat_ref[...].reshape(1, PB)
    seg = lax.broadcasted_iota(jnp.int32, (G, PB), 0)
    onehot = jnp.where(bat == seg, 1.0, 0.0)
    pt = lax.dot_general(onehot, h_ref[...], (((1,), (0,)), ((), ())),
                         preferred_element_type=jnp.float32)
    pc = jnp.sum(onehot, axis=1, keepdims=True)

    @pl.when(i == 0)
    def _():
        pooled[...] = jnp.zeros_like(pooled)
        cnt[...] = jnp.zeros_like(cnt)

    pooled[...] += pt
    cnt[...] += pc

    @pl.when(i == NPB - 1)
    def _():
        g = pooled[...] * (1.0 / jnp.maximum(cnt[...], 1.0))
        um = jnp.maximum(
            jnp.dot(g, w1m_ref[...], preferred_element_type=jnp.float32)
            + b1m_ref[...], 0.0)
        mu_ref[...] = jnp.maximum(
            jnp.dot(um, w2m_ref[...], preferred_element_type=jnp.float32)
            + b2m_ref[...], 0.0)
        uv = jnp.maximum(
            jnp.dot(g, w1v_ref[...], preferred_element_type=jnp.float32)
            + b1v_ref[...], 0.0)
        lv_ref[...] = jnp.maximum(
            jnp.dot(uv, w2v_ref[...], preferred_element_type=jnp.float32)
            + b2v_ref[...], 0.0)


def _pool_classify(h, batf, pm, pv):
    rowspec = pl.BlockSpec((PB, F), lambda i: (i, 0))
    batspec = pl.BlockSpec((1, 1, PB), lambda i: (i, 0, 0))
    mat = pl.BlockSpec((F, F), lambda i: (0, 0))
    vec = pl.BlockSpec((1, F), lambda i: (0, 0))
    gout = pl.BlockSpec((G, F), lambda i: (0, 0))
    return pl.pallas_call(
        _pool_body,
        grid=(NPB,),
        in_specs=[rowspec, batspec,
                  mat, vec, mat, vec, mat, vec, mat, vec],
        out_specs=[gout, gout],
        out_shape=[
            jax.ShapeDtypeStruct((G, F), jnp.float32),
            jax.ShapeDtypeStruct((G, F), jnp.float32),
        ],
        scratch_shapes=[
            pltpu.VMEM((G, F), jnp.float32),
            pltpu.VMEM((G, 1), jnp.float32),
        ],
    )(h, batf,
      pm["W1"], pm["b1"].reshape(1, F), pm["W2"], pm["b2"].reshape(1, F),
      pv["W1"], pv["b1"].reshape(1, F), pv["W2"], pv["b2"].reshape(1, F))


def kernel(x, edge_index, batch, params):
    src = edge_index[0].astype(jnp.int32).reshape(NW, CHUNKS, K)
    dst = edge_index[1].astype(jnp.int32).reshape(NW, CHUNKS, K)
    batf = batch.astype(jnp.int32).reshape(NPB, 1, PB)

    h = x
    for i in range(3):
        agg = _sc_agg(h, src, dst)
        h = _gin_layer(h, agg, params["convs"][i])

    class_mu, class_logv = _pool_classify(h, batf, params["cls_mu"],
                                          params["cls_logv"])

    agg = _sc_agg(h, src, dst)
    node_mu, node_logv = _gin_heads(h, agg, params["convs"][3],
                                    params["convs"][4])
    return (node_mu, node_logv, class_mu, class_logv)


# speedup vs baseline: 11.2420x; 1.0116x over previous
import functools

import jax
import jax.numpy as jnp
from jax import lax
from jax.experimental import pallas as pl
from jax.experimental.pallas import tpu as pltpu
from jax.experimental.pallas import tpu_sc as plsc

N = 10000
E = 320000
F = 128
G = 128
BN_EPS = 1e-5

NC = 2
NS = 16
NW = NC * NS

K = 125
CHUNKS = E // (NW * K)
SCH = 16
NSTG = CHUNKS // SCH
RPT = 624
TAIL = N - RPT * NS
ZCH = 52

BM = 2000
NB = N // BM
PB = 200
NPB = N // PB


def _sc_agg_body(h_hbm, src_hbm, dst_hbm, out_hbm, src_a, dst_a, src_b, dst_b,
                 rows_v, rows_b, zb_v, acc_sh, sem0, sem1, semz, semi):
    cid = lax.axis_index("c")
    sid = lax.axis_index("s")
    wid = sid * NC + cid
    rows = (rows_v, rows_b)
    sems = (sem0, sem1)
    idxs = ((src_a, dst_a), (src_b, dst_b))

    pltpu.sync_copy(src_hbm.at[wid, pl.ds(0, SCH)], src_a)
    pltpu.sync_copy(dst_hbm.at[wid, pl.ds(0, SCH)], dst_a)
    pltpu.async_copy(h_hbm.at[src_a.at[0]], rows[0], sems[0])
    pltpu.async_copy(h_hbm.at[src_a.at[1]], rows[1], sems[1])
    pltpu.async_copy(src_hbm.at[wid, pl.ds(SCH, SCH)], src_b, semi)
    pltpu.async_copy(dst_hbm.at[wid, pl.ds(SCH, SCH)], dst_b, semi)

    def _zero_row(i, _):
        for j in range(F // 16):
            zb_v[i, pl.ds(j * 16, 16)] = jnp.zeros((16,), jnp.float32)
        return 0
    lax.fori_loop(0, ZCH, _zero_row, 0)
    for r in range(RPT // ZCH):
        pltpu.async_copy(zb_v.at[pl.ds(0, ZCH)],
                         acc_sh.at[pl.ds(sid * RPT + r * ZCH, ZCH)], semz)

    @pl.when(sid == NS - 1)
    def _():
        pltpu.async_copy(zb_v.at[pl.ds(0, TAIL)],
                         acc_sh.at[pl.ds(N - TAIL, TAIL)], semz)

    for r in range(RPT // ZCH):
        pltpu.make_async_copy(zb_v.at[pl.ds(0, ZCH)],
                              acc_sh.at[pl.ds(sid * RPT, ZCH)], semz).wait()

    @pl.when(sid == NS - 1)
    def _():
        pltpu.make_async_copy(zb_v.at[pl.ds(0, TAIL)],
                              acc_sh.at[pl.ds(N - TAIL, TAIL)], semz).wait()

    plsc.subcore_barrier()

    for s in range(NSTG):
        sq, dq = idxs[s % 2]
        if 1 <= s < NSTG - 1:
            nsq, ndq = idxs[(s + 1) % 2]
            pltpu.async_copy(src_hbm.at[wid, pl.ds((s + 1) * SCH, SCH)],
                             nsq, semi)
            pltpu.async_copy(dst_hbm.at[wid, pl.ds((s + 1) * SCH, SCH)],
                             ndq, semi)

        def _pair(jj, _, sq=sq, dq=dq):
            j = 2 * jj
            for b in range(2):
                pltpu.make_async_copy(h_hbm.at[sq.at[0]], rows[b],
                                      sems[b]).wait()
                pltpu.sync_copy(rows[b], acc_sh.at[dq.at[j + b]], add=True)
                pltpu.async_copy(h_hbm.at[sq.at[j + b + 2]], rows[b], sems[b])
            return 0
        lax.fori_loop(0, SCH // 2 - 1, _pair, 0)

        for b in range(2):
            pltpu.make_async_copy(h_hbm.at[sq.at[0]], rows[b], sems[b]).wait()
            pltpu.sync_copy(rows[b], acc_sh.at[dq.at[SCH - 2 + b]], add=True)
            if s + 1 < NSTG:
                nsq, ndq = idxs[(s + 1) % 2]
                if b == 0:
                    pltpu.make_async_copy(
                        src_hbm.at[wid, pl.ds(0, SCH)], nsq, semi).wait()
                    pltpu.make_async_copy(
                        dst_hbm.at[wid, pl.ds(0, SCH)], ndq, semi).wait()
                pltpu.async_copy(h_hbm.at[nsq.at[b]], rows[b], sems[b])

    plsc.subcore_barrier()
    pltpu.sync_copy(acc_sh.at[pl.ds(sid * RPT, RPT)],
                    out_hbm.at[cid, pl.ds(sid * RPT, RPT)])

    @pl.when(sid == NS - 1)
    def _():
        pltpu.sync_copy(acc_sh.at[pl.ds(N - TAIL, TAIL)],
                        out_hbm.at[cid, pl.ds(N - TAIL, TAIL)])


@functools.lru_cache(maxsize=None)
def _get_sc_agg():
    return pl.kernel(
        _sc_agg_body,
        out_type=jax.ShapeDtypeStruct((NC, N, F), jnp.float32),
        mesh=plsc.VectorSubcoreMesh(core_axis_name="c", subcore_axis_name="s"),
        scratch_types=[
            pltpu.VMEM((SCH, K), jnp.int32),
            pltpu.VMEM((SCH, K), jnp.int32),
            pltpu.VMEM((SCH, K), jnp.int32),
            pltpu.VMEM((SCH, K), jnp.int32),
            pltpu.VMEM((K, F), jnp.float32),
            pltpu.VMEM((K, F), jnp.float32),
            pltpu.VMEM((ZCH, F), jnp.float32),
            pltpu.VMEM_SHARED((N, F), jnp.float32),
            pltpu.SemaphoreType.DMA,
            pltpu.SemaphoreType.DMA,
            pltpu.SemaphoreType.DMA,
            pltpu.SemaphoreType.DMA,
        ],
    )


def _sc_agg(h, src, dst):
    return _get_sc_agg()(h, src, dst)


def _gin_layer_body(h_ref, a0_ref, a1_ref, w1_ref, b1_ref, w2_ref, b2_ref,
                    g_ref, be_ref, o_ref, t_sc, s1_sc, s2_sc):
    ph = pl.program_id(0)
    i = pl.program_id(1)

    @pl.when(ph == 0)
    def _():
        z = h_ref[...] + a0_ref[...] + a1_ref[...]
        u = jnp.maximum(
            jnp.dot(z, w1_ref[...], preferred_element_type=jnp.float32)
            + b1_ref[...], 0.0)
        t = jnp.maximum(
            jnp.dot(u, w2_ref[...], preferred_element_type=jnp.float32)
            + b2_ref[...], 0.0)
        t_sc[pl.ds(i * BM, BM), :] = t
        mb = jnp.mean(t, axis=0, keepdims=True)
        m2b = jnp.sum((t - mb) * (t - mb), axis=0, keepdims=True)

        @pl.when(i == 0)
        def _():
            s1_sc[...] = mb
            s2_sc[...] = m2b

        @pl.when(i > 0)
        def _():
            na = i.astype(jnp.float32) * BM
            n = na + BM
            delta = mb - s1_sc[...]
            s1_sc[...] += delta * (BM / n)
            s2_sc[...] += m2b + delta * delta * (na * BM / n)

    @pl.when(ph == 1)
    def _():
        a = g_ref[...] * lax.rsqrt(s2_sc[...] / N + BN_EPS)
        c = be_ref[...] - s1_sc[...] * a
        o_ref[...] = t_sc[pl.ds(i * BM, BM), :] * a + c


def _gin_layer(h, agg, p):
    rowin = pl.BlockSpec((BM, F), lambda ph, i: (i * (1 - ph), 0))
    mat = pl.BlockSpec((F, F), lambda ph, i: (0, 0))
    vec = pl.BlockSpec((1, F), lambda ph, i: (0, 0))
    return pl.pallas_call(
        _gin_layer_body,
        grid=(2, NB),
        in_specs=[rowin, rowin, rowin, mat, vec, mat, vec, vec, vec],
        out_specs=pl.BlockSpec((BM, F), lambda ph, i: (i * ph, 0)),
        out_shape=jax.ShapeDtypeStruct((N, F), jnp.float32),
        scratch_shapes=[
            pltpu.VMEM((N, F), jnp.float32),
            pltpu.VMEM((1, F), jnp.float32),
            pltpu.VMEM((1, F), jnp.float32),
        ],
    )(h, agg[0], agg[1], p["W1"], p["b1"].reshape(1, F), p["W2"],
      p["b2"].reshape(1, F), p["gamma"].reshape(1, F), p["beta"].reshape(1, F))


def _gin_heads_body(h_ref, a0_ref, a1_ref,
                    w1m_ref, b1m_ref, w2m_ref, b2m_ref, gm_ref, bm_ref,
                    w1v_ref, b1v_ref, w2v_ref, b2v_ref, gv_ref, bv_ref,
                    om_ref, ov_ref,
                    tm_sc, tv_sc, s1m_sc, s2m_sc, s1v_sc, s2v_sc):
    ph = pl.program_id(0)
    i = pl.program_id(1)

    @pl.when(ph == 0)
    def _():
        z = h_ref[...] + a0_ref[...] + a1_ref[...]
        for (w1, b1, w2, b2, t_sc, s1_sc, s2_sc) in (
                (w1m_ref, b1m_ref, w2m_ref, b2m_ref, tm_sc, s1m_sc, s2m_sc),
                (w1v_ref, b1v_ref, w2v_ref, b2v_ref, tv_sc, s1v_sc, s2v_sc)):
            u = jnp.maximum(
                jnp.dot(z, w1[...], preferred_element_type=jnp.float32)
                + b1[...], 0.0)
            t = jnp.maximum(
                jnp.dot(u, w2[...], preferred_element_type=jnp.float32)
                + b2[...], 0.0)
            t_sc[pl.ds(i * BM, BM), :] = t
            mb = jnp.mean(t, axis=0, keepdims=True)
            m2b = jnp.sum((t - mb) * (t - mb), axis=0, keepdims=True)

            @pl.when(i == 0)
            def _():
                s1_sc[...] = mb
                s2_sc[...] = m2b

            @pl.when(i > 0)
            def _():
                na = i.astype(jnp.float32) * BM
                n = na + BM
                delta = mb - s1_sc[...]
                s1_sc[...] += delta * (BM / n)
                s2_sc[...] += m2b + delta * delta * (na * BM / n)

    @pl.when(ph == 1)
    def _():
        for (g_ref, be_ref, t_sc, s1_sc, s2_sc, o_ref) in (
                (gm_ref, bm_ref, tm_sc, s1m_sc, s2m_sc, om_ref),
                (gv_ref, bv_ref, tv_sc, s1v_sc, s2v_sc, ov_ref)):
            a = g_ref[...] * lax.rsqrt(s2_sc[...] / N + BN_EPS)
            c = be_ref[...] - s1_sc[...] * a
            o_ref[...] = t_sc[pl.ds(i * BM, BM), :] * a + c


def _gin_heads(h, agg, pm, pv):
    rowin = pl.BlockSpec((BM, F), lambda ph, i: (i * (1 - ph), 0))
    rowout = pl.BlockSpec((BM, F), lambda ph, i: (i * ph, 0))
    mat = pl.BlockSpec((F, F), lambda ph, i: (0, 0))
    vec = pl.BlockSpec((1, F), lambda ph, i: (0, 0))
    return pl.pallas_call(
        _gin_heads_body,
        grid=(2, NB),
        in_specs=[rowin, rowin, rowin,
                  mat, vec, mat, vec, vec, vec,
                  mat, vec, mat, vec, vec, vec],
        out_specs=[rowout, rowout],
        out_shape=[
            jax.ShapeDtypeStruct((N, F), jnp.float32),
            jax.ShapeDtypeStruct((N, F), jnp.float32),
        ],
        scratch_shapes=[
            pltpu.VMEM((N, F), jnp.float32),
            pltpu.VMEM((N, F), jnp.float32),
            pltpu.VMEM((1, F), jnp.float32),
            pltpu.VMEM((1, F), jnp.float32),
            pltpu.VMEM((1, F), jnp.float32),
            pltpu.VMEM((1, F), jnp.float32),
        ],
    )(h, agg[0], agg[1],
      pm["W1"], pm["b1"].reshape(1, F), pm["W2"], pm["b2"].reshape(1, F),
      pm["gamma"].reshape(1, F), pm["beta"].reshape(1, F),
      pv["W1"], pv["b1"].reshape(1, F), pv["W2"], pv["b2"].reshape(1, F),
      pv["gamma"].reshape(1, F), pv["beta"].reshape(1, F))


def _pool_body(h_ref, bat_ref, w1m_ref, b1m_ref, w2m_ref, b2m_ref,
               w1v_ref, b1v_ref, w2v_ref, b2v_ref,
               mu_ref, lv_ref, pooled, cnt):
    i = pl.program_id(0)
    bat = bat_ref[...].reshape(1, PB)
    seg = lax.broadcasted_iota(jnp.int32, (G, PB), 0)
    onehot = jnp.where(bat == seg, 1.0, 0.0)
    pt = lax.dot_general(onehot, h_ref[...], (((1,), (0,)), ((), ())),
                         preferred_element_type=jnp.float32)
    pc = jnp.sum(onehot, axis=1, keepdims=True)

    @pl.when(i == 0)
    def _():
        pooled[...] = jnp.zeros_like(pooled)
        cnt[...] = jnp.zeros_like(cnt)

    pooled[...] += pt
    cnt[...] += pc

    @pl.when(i == NPB - 1)
    def _():
        g = pooled[...] * (1.0 / jnp.maximum(cnt[...], 1.0))
        um = jnp.maximum(
            jnp.dot(g, w1m_ref[...], preferred_element_type=jnp.float32)
            + b1m_ref[...], 0.0)
        mu_ref[...] = jnp.maximum(
            jnp.dot(um, w2m_ref[...], preferred_element_type=jnp.float32)
            + b2m_ref[...], 0.0)
        uv = jnp.maximum(
            jnp.dot(g, w1v_ref[...], preferred_element_type=jnp.float32)
            + b1v_ref[...], 0.0)
        lv_ref[...] = jnp.maximum(
            jnp.dot(uv, w2v_ref[...], preferred_element_type=jnp.float32)
            + b2v_ref[...], 0.0)


def _pool_classify(h, batf, pm, pv):
    rowspec = pl.BlockSpec((PB, F), lambda i: (i, 0))
    batspec = pl.BlockSpec((1, 1, PB), lambda i: (i, 0, 0))
    mat = pl.BlockSpec((F, F), lambda i: (0, 0))
    vec = pl.BlockSpec((1, F), lambda i: (0, 0))
    gout = pl.BlockSpec((G, F), lambda i: (0, 0))
    return pl.pallas_call(
        _pool_body,
        grid=(NPB,),
        in_specs=[rowspec, batspec,
                  mat, vec, mat, vec, mat, vec, mat, vec],
        out_specs=[gout, gout],
        out_shape=[
            jax.ShapeDtypeStruct((G, F), jnp.float32),
            jax.ShapeDtypeStruct((G, F), jnp.float32),
        ],
        scratch_shapes=[
            pltpu.VMEM((G, F), jnp.float32),
            pltpu.VMEM((G, 1), jnp.float32),
        ],
    )(h, batf,
      pm["W1"], pm["b1"].reshape(1, F), pm["W2"], pm["b2"].reshape(1, F),
      pv["W1"], pv["b1"].reshape(1, F), pv["W2"], pv["b2"].reshape(1, F))


def kernel(x, edge_index, batch, params):
    src = edge_index[0].astype(jnp.int32).reshape(NW, CHUNKS, K)
    dst = edge_index[1].astype(jnp.int32).reshape(NW, CHUNKS, K)
    batf = batch.astype(jnp.int32).reshape(NPB, 1, PB)

    h = x
    for i in range(3):
        agg = _sc_agg(h, src, dst)
        h = _gin_layer(h, agg, params["convs"][i])

    class_mu, class_logv = _pool_classify(h, batf, params["cls_mu"],
                                          params["cls_logv"])

    agg = _sc_agg(h, src, dst)
    node_mu, node_logv = _gin_heads(h, agg, params["convs"][3],
                                    params["convs"][4])
    return (node_mu, node_logv, class_mu, class_logv)


# speedup vs baseline: 11.2529x; 1.0010x over previous
import functools

import jax
import jax.numpy as jnp
from jax import lax
from jax.experimental import pallas as pl
from jax.experimental.pallas import tpu as pltpu
from jax.experimental.pallas import tpu_sc as plsc

N = 10000
E = 320000
F = 128
G = 128
BN_EPS = 1e-5

NC = 2
NS = 16
NW = NC * NS

K = 125
CHUNKS = E // (NW * K)
SCH = 16
NSTG = CHUNKS // SCH
RPT = 624
TAIL = N - RPT * NS
ZCH = 52

BM = 2000
NB = N // BM
PB = 200
NPB = N // PB


def _sc_agg_body(h_hbm, src_hbm, dst_hbm, out_hbm, src_a, dst_a, src_b, dst_b,
                 rows_v, rows_b, zb_v, acc_sh, sem0, sem1, semz, semi):
    cid = lax.axis_index("c")
    sid = lax.axis_index("s")
    wid = sid * NC + cid
    rows = (rows_v, rows_b)
    sems = (sem0, sem1)
    idxs = ((src_a, dst_a), (src_b, dst_b))

    pltpu.sync_copy(src_hbm.at[wid, pl.ds(0, SCH)], src_a)
    pltpu.sync_copy(dst_hbm.at[wid, pl.ds(0, SCH)], dst_a)
    pltpu.async_copy(h_hbm.at[src_a.at[0]], rows[0], sems[0])
    pltpu.async_copy(h_hbm.at[src_a.at[1]], rows[1], sems[1])
    pltpu.async_copy(src_hbm.at[wid, pl.ds(SCH, SCH)], src_b, semi)
    pltpu.async_copy(dst_hbm.at[wid, pl.ds(SCH, SCH)], dst_b, semi)

    def _zero_row(i, _):
        for j in range(F // 16):
            zb_v[i, pl.ds(j * 16, 16)] = jnp.zeros((16,), jnp.float32)
        return 0
    lax.fori_loop(0, ZCH, _zero_row, 0)
    for r in range(RPT // ZCH):
        pltpu.async_copy(zb_v.at[pl.ds(0, ZCH)],
                         acc_sh.at[pl.ds(sid * RPT + r * ZCH, ZCH)], semz)

    @pl.when(sid == NS - 1)
    def _():
        pltpu.async_copy(zb_v.at[pl.ds(0, TAIL)],
                         acc_sh.at[pl.ds(N - TAIL, TAIL)], semz)

    for r in range(RPT // ZCH):
        pltpu.make_async_copy(zb_v.at[pl.ds(0, ZCH)],
                              acc_sh.at[pl.ds(sid * RPT, ZCH)], semz).wait()

    @pl.when(sid == NS - 1)
    def _():
        pltpu.make_async_copy(zb_v.at[pl.ds(0, TAIL)],
                              acc_sh.at[pl.ds(N - TAIL, TAIL)], semz).wait()

    plsc.subcore_barrier()

    for s in range(NSTG):
        sq, dq = idxs[s % 2]
        if 1 <= s < NSTG - 1:
            nsq, ndq = idxs[(s + 1) % 2]
            pltpu.async_copy(src_hbm.at[wid, pl.ds((s + 1) * SCH, SCH)],
                             nsq, semi)
            pltpu.async_copy(dst_hbm.at[wid, pl.ds((s + 1) * SCH, SCH)],
                             ndq, semi)

        def _pair(jj, _, sq=sq, dq=dq):
            j = 2 * jj
            for b in range(2):
                pltpu.make_async_copy(h_hbm.at[sq.at[0]], rows[b],
                                      sems[b]).wait()
                pltpu.sync_copy(rows[b], acc_sh.at[dq.at[j + b]], add=True)
                pltpu.async_copy(h_hbm.at[sq.at[j + b + 2]], rows[b], sems[b])
            return 0
        lax.fori_loop(0, SCH // 2 - 1, _pair, 0)

        for b in range(2):
            pltpu.make_async_copy(h_hbm.at[sq.at[0]], rows[b], sems[b]).wait()
            pltpu.sync_copy(rows[b], acc_sh.at[dq.at[SCH - 2 + b]], add=True)
            if s + 1 < NSTG:
                nsq, ndq = idxs[(s + 1) % 2]
                if b == 0:
                    pltpu.make_async_copy(
                        src_hbm.at[wid, pl.ds(0, SCH)], nsq, semi).wait()
                    pltpu.make_async_copy(
                        dst_hbm.at[wid, pl.ds(0, SCH)], ndq, semi).wait()
                pltpu.async_copy(h_hbm.at[nsq.at[b]], rows[b], sems[b])

    plsc.subcore_barrier()
    pltpu.sync_copy(acc_sh.at[pl.ds(sid * RPT, RPT)],
                    out_hbm.at[cid, pl.ds(sid * RPT, RPT)])

    @pl.when(sid == NS - 1)
    def _():
        pltpu.sync_copy(acc_sh.at[pl.ds(N - TAIL, TAIL)],
                        out_hbm.at[cid, pl.ds(N - TAIL, TAIL)])


@functools.lru_cache(maxsize=None)
def _get_sc_agg():
    return pl.kernel(
        _sc_agg_body,
        out_type=jax.ShapeDtypeStruct((NC, N, F), jnp.float32),
        mesh=plsc.VectorSubcoreMesh(core_axis_name="c", subcore_axis_name="s"),
        scratch_types=[
            pltpu.VMEM((SCH, K), jnp.int32),
            pltpu.VMEM((SCH, K), jnp.int32),
            pltpu.VMEM((SCH, K), jnp.int32),
            pltpu.VMEM((SCH, K), jnp.int32),
            pltpu.VMEM((K, F), jnp.float32),
            pltpu.VMEM((K, F), jnp.float32),
            pltpu.VMEM((ZCH, F), jnp.float32),
            pltpu.VMEM_SHARED((N, F), jnp.float32),
            pltpu.SemaphoreType.DMA,
            pltpu.SemaphoreType.DMA,
            pltpu.SemaphoreType.DMA,
            pltpu.SemaphoreType.DMA,
        ],
    )


def _sc_agg(h, src, dst):
    return _get_sc_agg()(h, src, dst)


def _gin_layer_body(h_ref, a0_ref, a1_ref, w1_ref, b1_ref, w2_ref, b2_ref,
                    g_ref, be_ref, o_ref, t_sc, s1_sc, s2_sc):
    ph = pl.program_id(0)
    i = pl.program_id(1)

    @pl.when(ph == 0)
    def _():
        z = h_ref[...] + a0_ref[...] + a1_ref[...]
        u = jnp.maximum(
            jnp.dot(z, w1_ref[...], preferred_element_type=jnp.float32)
            + b1_ref[...], 0.0)
        t = jnp.maximum(
            jnp.dot(u, w2_ref[...], preferred_element_type=jnp.float32)
            + b2_ref[...], 0.0)
        t_sc[pl.ds(i * BM, BM), :] = t
        mb = jnp.mean(t, axis=0, keepdims=True)
        m2b = jnp.sum((t - mb) * (t - mb), axis=0, keepdims=True)

        @pl.when(i == 0)
        def _():
            s1_sc[...] = mb
            s2_sc[...] = m2b

        @pl.when(i > 0)
        def _():
            na = i.astype(jnp.float32) * BM
            n = na + BM
            delta = mb - s1_sc[...]
            s1_sc[...] += delta * (BM / n)
            s2_sc[...] += m2b + delta * delta * (na * BM / n)

    @pl.when(ph == 1)
    def _():
        a = g_ref[...] * lax.rsqrt(s2_sc[...] / N + BN_EPS)
        c = be_ref[...] - s1_sc[...] * a
        o_ref[...] = t_sc[pl.ds(i * BM, BM), :] * a + c


def _gin_layer(h, agg, p):
    rowin = pl.BlockSpec((BM, F), lambda ph, i: (i * (1 - ph) + (NB - 1) * ph, 0))
    mat = pl.BlockSpec((F, F), lambda ph, i: (0, 0))
    vec = pl.BlockSpec((1, F), lambda ph, i: (0, 0))
    return pl.pallas_call(
        _gin_layer_body,
        grid=(2, NB),
        in_specs=[rowin, rowin, rowin, mat, vec, mat, vec, vec, vec],
        out_specs=pl.BlockSpec((BM, F), lambda ph, i: (i * ph, 0)),
        out_shape=jax.ShapeDtypeStruct((N, F), jnp.float32),
        scratch_shapes=[
            pltpu.VMEM((N, F), jnp.float32),
            pltpu.VMEM((1, F), jnp.float32),
            pltpu.VMEM((1, F), jnp.float32),
        ],
    )(h, agg[0], agg[1], p["W1"], p["b1"].reshape(1, F), p["W2"],
      p["b2"].reshape(1, F), p["gamma"].reshape(1, F), p["beta"].reshape(1, F))


def _gin_heads_body(h_ref, a0_ref, a1_ref,
                    w1m_ref, b1m_ref, w2m_ref, b2m_ref, gm_ref, bm_ref,
                    w1v_ref, b1v_ref, w2v_ref, b2v_ref, gv_ref, bv_ref,
                    om_ref, ov_ref,
                    tm_sc, tv_sc, s1m_sc, s2m_sc, s1v_sc, s2v_sc):
    ph = pl.program_id(0)
    i = pl.program_id(1)

    @pl.when(ph == 0)
    def _():
        z = h_ref[...] + a0_ref[...] + a1_ref[...]
        for (w1, b1, w2, b2, t_sc, s1_sc, s2_sc) in (
                (w1m_ref, b1m_ref, w2m_ref, b2m_ref, tm_sc, s1m_sc, s2m_sc),
                (w1v_ref, b1v_ref, w2v_ref, b2v_ref, tv_sc, s1v_sc, s2v_sc)):
            u = jnp.maximum(
                jnp.dot(z, w1[...], preferred_element_type=jnp.float32)
                + b1[...], 0.0)
            t = jnp.maximum(
                jnp.dot(u, w2[...], preferred_element_type=jnp.float32)
                + b2[...], 0.0)
            t_sc[pl.ds(i * BM, BM), :] = t
            mb = jnp.mean(t, axis=0, keepdims=True)
            m2b = jnp.sum((t - mb) * (t - mb), axis=0, keepdims=True)

            @pl.when(i == 0)
            def _():
                s1_sc[...] = mb
                s2_sc[...] = m2b

            @pl.when(i > 0)
            def _():
                na = i.astype(jnp.float32) * BM
                n = na + BM
                delta = mb - s1_sc[...]
                s1_sc[...] += delta * (BM / n)
                s2_sc[...] += m2b + delta * delta * (na * BM / n)

    @pl.when(ph == 1)
    def _():
        for (g_ref, be_ref, t_sc, s1_sc, s2_sc, o_ref) in (
                (gm_ref, bm_ref, tm_sc, s1m_sc, s2m_sc, om_ref),
                (gv_ref, bv_ref, tv_sc, s1v_sc, s2v_sc, ov_ref)):
            a = g_ref[...] * lax.rsqrt(s2_sc[...] / N + BN_EPS)
            c = be_ref[...] - s1_sc[...] * a
            o_ref[...] = t_sc[pl.ds(i * BM, BM), :] * a + c


def _gin_heads(h, agg, pm, pv):
    rowin = pl.BlockSpec((BM, F), lambda ph, i: (i * (1 - ph) + (NB - 1) * ph, 0))
    rowout = pl.BlockSpec((BM, F), lambda ph, i: (i * ph, 0))
    mat = pl.BlockSpec((F, F), lambda ph, i: (0, 0))
    vec = pl.BlockSpec((1, F), lambda ph, i: (0, 0))
    return pl.pallas_call(
        _gin_heads_body,
        grid=(2, NB),
        in_specs=[rowin, rowin, rowin,
                  mat, vec, mat, vec, vec, vec,
                  mat, vec, mat, vec, vec, vec],
        out_specs=[rowout, rowout],
        out_shape=[
            jax.ShapeDtypeStruct((N, F), jnp.float32),
            jax.ShapeDtypeStruct((N, F), jnp.float32),
        ],
        scratch_shapes=[
            pltpu.VMEM((N, F), jnp.float32),
            pltpu.VMEM((N, F), jnp.float32),
            pltpu.VMEM((1, F), jnp.float32),
            pltpu.VMEM((1, F), jnp.float32),
            pltpu.VMEM((1, F), jnp.float32),
            pltpu.VMEM((1, F), jnp.float32),
        ],
    )(h, agg[0], agg[1],
      pm["W1"], pm["b1"].reshape(1, F), pm["W2"], pm["b2"].reshape(1, F),
      pm["gamma"].reshape(1, F), pm["beta"].reshape(1, F),
      pv["W1"], pv["b1"].reshape(1, F), pv["W2"], pv["b2"].reshape(1, F),
      pv["gamma"].reshape(1, F), pv["beta"].reshape(1, F))


def _pool_body(h_ref, bat_ref, w1m_ref, b1m_ref, w2m_ref, b2m_ref,
               w1v_ref, b1v_ref, w2v_ref, b2v_ref,
               mu_ref, lv_ref, pooled, cnt):
    i = pl.program_id(0)
    bat = bat_ref[...].reshape(1, PB)
    seg = lax.broadcasted_iota(jnp.int32, (G, PB), 0)
    onehot = jnp.where(bat == seg, 1.0, 0.0)
    pt = lax.dot_general(onehot, h_ref[...], (((1,), (0,)), ((), ())),
                         preferred_element_type=jnp.float32)
    pc = jnp.sum(onehot, axis=1, keepdims=True)

    @pl.when(i == 0)
    def _():
        pooled[...] = jnp.zeros_like(pooled)
        cnt[...] = jnp.zeros_like(cnt)

    pooled[...] += pt
    cnt[...] += pc

    @pl.when(i == NPB - 1)
    def _():
        g = pooled[...] * (1.0 / jnp.maximum(cnt[...], 1.0))
        um = jnp.maximum(
            jnp.dot(g, w1m_ref[...], preferred_element_type=jnp.float32)
            + b1m_ref[...], 0.0)
        mu_ref[...] = jnp.maximum(
            jnp.dot(um, w2m_ref[...], preferred_element_type=jnp.float32)
            + b2m_ref[...], 0.0)
        uv = jnp.maximum(
            jnp.dot(g, w1v_ref[...], preferred_element_type=jnp.float32)
            + b1v_ref[...], 0.0)
        lv_ref[...] = jnp.maximum(
            jnp.dot(uv, w2v_ref[...], preferred_element_type=jnp.float32)
            + b2v_ref[...], 0.0)


def _pool_classify(h, batf, pm, pv):
    rowspec = pl.BlockSpec((PB, F), lambda i: (i, 0))
    batspec = pl.BlockSpec((1, 1, PB), lambda i: (i, 0, 0))
    mat = pl.BlockSpec((F, F), lambda i: (0, 0))
    vec = pl.BlockSpec((1, F), lambda i: (0, 0))
    gout = pl.BlockSpec((G, F), lambda i: (0, 0))
    return pl.pallas_call(
        _pool_body,
        grid=(NPB,),
        in_specs=[rowspec, batspec,
                  mat, vec, mat, vec, mat, vec, mat, vec],
        out_specs=[gout, gout],
        out_shape=[
            jax.ShapeDtypeStruct((G, F), jnp.float32),
            jax.ShapeDtypeStruct((G, F), jnp.float32),
        ],
        scratch_shapes=[
            pltpu.VMEM((G, F), jnp.float32),
            pltpu.VMEM((G, 1), jnp.float32),
        ],
    )(h, batf,
      pm["W1"], pm["b1"].reshape(1, F), pm["W2"], pm["b2"].reshape(1, F),
      pv["W1"], pv["b1"].reshape(1, F), pv["W2"], pv["b2"].reshape(1, F))


def kernel(x, edge_index, batch, params):
    src = edge_index[0].astype(jnp.int32).reshape(NW, CHUNKS, K)
    dst = edge_index[1].astype(jnp.int32).reshape(NW, CHUNKS, K)
    batf = batch.astype(jnp.int32).reshape(NPB, 1, PB)

    h = x
    for i in range(3):
        agg = _sc_agg(h, src, dst)
        h = _gin_layer(h, agg, params["convs"][i])

    class_mu, class_logv = _pool_classify(h, batf, params["cls_mu"],
                                          params["cls_logv"])

    agg = _sc_agg(h, src, dst)
    node_mu, node_logv = _gin_heads(h, agg, params["convs"][3],
                                    params["convs"][4])
    return (node_mu, node_logv, class_mu, class_logv)
